# baseline (speedup 1.0000x reference)
import jax
import jax.numpy as jnp
from jax.experimental import pallas as pl
from jax.experimental.pallas import tpu as pltpu

B = 32
C = 39
L = 2048
D = 256
H = 256
K = 6
STRIDE = 2
NCODES = 1024
NHEADS = 4
ATT_H = 128

T = (L - K) // STRIDE + 1
TP = 1024
CK = C * STRIDE
CKP = 128

TCH = 64
NC = TP // TCH

F32 = jnp.float32
BF16 = jnp.bfloat16


def _split_lhs(a):
    ah = a.astype(BF16)
    al = (a - ah.astype(F32)).astype(BF16)
    return ah, al


def _mm3(a, wh, wl):
    ah, al = _split_lhs(a)
    return (jnp.dot(ah, wh, preferred_element_type=F32)
            + jnp.dot(al, wh, preferred_element_type=F32)
            + jnp.dot(ah, wl, preferred_element_type=F32))


def _mm3p(ah, al, wh, wl):
    return (jnp.dot(ah, wh, preferred_element_type=F32)
            + jnp.dot(al, wh, preferred_element_type=F32)
            + jnp.dot(ah, wl, preferred_element_type=F32))


def _gru_gates(g, gh, h):
    r = jax.nn.sigmoid(g[:, :H] + gh[:, :H])
    z = jax.nn.sigmoid(g[:, H:2 * H] + gh[:, H:2 * H])
    n = jnp.tanh(g[:, 2 * H:] + r * gh[:, 2 * H:])
    return (1.0 - z) * n + z * h


def _fused(l_ref, a0h_ref, a0l_ref, a1h_ref, a1l_ref, a2h_ref, a2l_ref,
           w0h_ref, w0l_ref, w1h_ref, w1l_ref, w2h_ref, w2l_ref, cb_ref,
           wihh_ref, wihl_ref, bih_ref, whhh_ref, whhl_ref, bhh_ref,
           cbkh_ref, cbkl_ref, cbkT_ref, cbTh_ref, cbTl_ref,
           wih2h_ref, wih2l_ref, bih2_ref, whh2h_ref, whh2l_ref, bhh2_ref,
           wfh_ref, wfl_ref, bf_ref, uf_ref,
           out_ref,
           h1_ref, h2_ref, pidx_ref, gi1_ref, gi2_ref, y_ref, hs_ref,
           cont_ref, idx_ref, n0_ref, n1_ref, n2_ref, n3_ref, den_ref,
           scst_ref, hsst_ref, cn_ref):
    s = pl.program_id(0)
    nums = [n0_ref, n1_ref, n2_ref, n3_ref]

    @pl.when(s == 0)
    def _():
        h1_ref[...] = jnp.zeros_like(h1_ref)
        cbT = cbkT_ref[...]
        cn_ref[0:1] = jnp.sum(cbT * cbT, axis=0, keepdims=True)

    @pl.when(s == 1)
    def _():
        h2_ref[...] = jnp.zeros_like(h2_ref)
        pidx_ref[...] = jnp.full_like(pidx_ref, -1)
        for nr in nums:
            nr[...] = jnp.zeros_like(nr)
        den_ref[...] = jnp.zeros_like(den_ref)

    x = (_mm3p(a0h_ref[...].reshape(TCH * B, CKP),
               a0l_ref[...].reshape(TCH * B, CKP), w0h_ref[...], w0l_ref[...])
         + _mm3p(a1h_ref[...].reshape(TCH * B, CKP),
                 a1l_ref[...].reshape(TCH * B, CKP), w1h_ref[...],
                 w1l_ref[...])
         + _mm3p(a2h_ref[...].reshape(TCH * B, CKP),
                 a2l_ref[...].reshape(TCH * B, CKP), w2h_ref[...],
                 w2l_ref[...])
         + cb_ref[...])
    gi1 = _mm3(x, wihh_ref[...], wihl_ref[...]) + bih_ref[...]
    gi1_ref[...] = gi1.reshape(TCH, B, 3 * H)

    @pl.when(s >= 1)
    def _():
        yf = y_ref[...].reshape(TCH * B, H)
        d = cn_ref[0:1] - 2.0 * _mm3(yf, cbTh_ref[...], cbTl_ref[...])
        d3 = d.reshape(TCH, B, NCODES)
        dmin = jnp.min(d3, axis=2, keepdims=True)
        iota = jax.lax.broadcasted_iota(jnp.int32, (TCH, B, NCODES), 2)
        cand = jnp.where(d3 == dmin, iota, NCODES)
        cmin = jnp.min(cand, axis=2, keepdims=True)
        idx3 = cmin[:, :, 0]
        idx_ref[...] = idx3
        onehot = (cand == cmin).astype(BF16).reshape(TCH * B, NCODES)
        q = (jnp.dot(onehot, cbkh_ref[...], preferred_element_type=F32)
             + jnp.dot(onehot, cbkl_ref[...], preferred_element_type=F32))
        gi2 = _mm3(q, wih2h_ref[...], wih2l_ref[...]) + bih2_ref[...]
        gi2_ref[...] = gi2.reshape(TCH, B, 3 * H)

        @pl.when(s >= 2)
        def _():
            nsd = (idx3[0:1] != pidx_ref[0:1]).astype(F32)
            hstash = hsst_ref[...]
            for k in range(NHEADS):
                wd = jnp.exp(scst_ref[k]) * nsd[0]
                nums[k][...] += wd[:, None] * hstash
                den_ref[k] += wd
            den_ref[NHEADS] += nsd[0]

        prev = jnp.concatenate([pidx_ref[0:1], idx3[:TCH - 1]], axis=0)
        cont3 = (idx3 == prev).astype(F32)[:, :, None]
        cont_ref[...] = jnp.broadcast_to(cont3, (TCH, B, H))
        pidx_ref[0:1] = idx3[TCH - 1:]

    whh1h = whhh_ref[...]
    whh1l = whhl_ref[...]
    bhh1 = bhh_ref[...]
    whh2h = whh2h_ref[...]
    whh2l = whh2l_ref[...]
    bhh2 = bhh2_ref[...]

    def body(k, carry):
        h1, h2 = carry
        g1 = gi1_ref[pl.ds(k, 1)][0]
        gh1 = _mm3(h1, whh1h, whh1l) + bhh1
        hn1 = _gru_gates(g1, gh1, h1)
        y_ref[pl.ds(k, 1)] = hn1[None]

        g2 = gi2_ref[pl.ds(k, 1)][0]
        c2 = cont_ref[pl.ds(k, 1)][0]
        hin = h2 * c2
        gh2 = _mm3(hin, whh2h, whh2l) + bhh2
        hn2 = _gru_gates(g2, gh2, hin)
        hs_ref[pl.ds(k, 1)] = hn2[None]
        return hn1, hn2

    h1f, h2f = jax.lax.fori_loop(0, TCH, body,
                                 (h1_ref[...], h2_ref[...]))
    h1_ref[...] = h1f
    h2_ref[...] = h2f

    lc3 = ((l_ref[...] - (K - STRIDE)) // STRIDE)[None, :, :]
    tt3 = jax.lax.broadcasted_iota(jnp.int32, (TCH, B, 1), 0) + s * TCH
    y_ref[...] = y_ref[...] * (tt3 < lc3).astype(F32)

    @pl.when(s >= 1)
    def _():
        idx3 = idx_ref[...]
        hs3 = hs_ref[...]
        hsf = hs3.reshape(TCH * B, H)
        e = jnp.tanh(_mm3(hsf, wfh_ref[...], wfl_ref[...]) + bf_ref[...])
        ew3 = (e * uf_ref[...]).reshape(TCH, B, NHEADS * ATT_H)
        tglob = (jax.lax.broadcasted_iota(jnp.int32, (TCH, B), 0)
                 + (s - 1) * TCH)
        nxt = jnp.concatenate([idx3[1:], idx3[TCH - 1:]], axis=0)
        nsr = jnp.logical_or(nxt != idx3, tglob == T - 1)
        m3 = jnp.logical_and(nsr, tglob < T).astype(F32)
        for k in range(NHEADS):
            sk = jnp.sum(ew3[:, :, k * ATT_H:(k + 1) * ATT_H], axis=2)
            wk = jnp.exp(sk) * m3
            nums[k][...] += jnp.sum(wk[:, :, None] * hs3, axis=0)
            den_ref[k] += jnp.sum(wk, axis=0)
            scst_ref[k:k + 1] = sk[TCH - 1:]
        den_ref[NHEADS] += jnp.sum(m3, axis=0)
        hsst_ref[...] = hs3[TCH - 1]

    @pl.when(s == NC)
    def _():
        ew0 = jnp.tanh(bf_ref[...]) * uf_ref[...]
        nseg = den_ref[NHEADS]
        ss = jnp.zeros((B, 1), F32)
        ps = []
        for k in range(NHEADS):
            s0 = jnp.sum(ew0[0, k * ATT_H:(k + 1) * ATT_H])
            dk = den_ref[k] + (float(T) - nseg) * jnp.exp(s0)
            pk = nums[k][...] / dk[:, None]
            ps.append(pk)
            ss = ss + jnp.sum(pk * pk, axis=1, keepdims=True)
        nrm = jnp.maximum(jnp.sqrt(ss), 1e-12)
        for k in range(NHEADS):
            out_ref[:, k * H:(k + 1) * H] = ps[k] / nrm


def kernel(input, l, conv_w, conv_b, gru_Wih, gru_Whh, gru_bih, gru_bhh,
           cell_Wih, cell_Whh, cell_bih, cell_bhh, codebook, att_W, att_b,
           att_u):
    p = input.reshape(B, C, L // STRIDE, STRIDE)
    p = p.transpose(2, 0, 1, 3).reshape(L // STRIDE, B, CK)
    p = jnp.pad(p, ((0, 0), (0, 0), (0, CKP - CK)))
    a0 = jnp.pad(p, ((0, TCH), (0, 0), (0, 0)))
    a1 = jnp.pad(p[1:], ((0, TCH + 1), (0, 0), (0, 0)))
    a2 = jnp.pad(p[2:], ((0, TCH + 2), (0, 0), (0, 0)))
    a0h, a0l = _split_lhs(a0)
    a1h, a1l = _split_lhs(a1)
    a2h, a2l = _split_lhs(a2)
    wsp = []
    for j in range(K // STRIDE):
        wj = conv_w[:, :, STRIDE * j:STRIDE * (j + 1)].reshape(D, CK).T
        wsp.extend(_split_lhs(jnp.pad(wj, ((0, CKP - CK), (0, 0)))))
    w0h, w0l, w1h, w1l, w2h, w2l = wsp
    l2 = l.astype(jnp.int32).reshape(B, 1)
    cb2 = conv_b.reshape(1, D)
    wihh, wihl = _split_lhs(gru_Wih.T)
    whhh, whhl = _split_lhs(gru_Whh.T)
    bih2 = gru_bih.reshape(1, 3 * H)
    bhh2 = gru_bhh.reshape(1, 3 * H)
    wih2h, wih2l = _split_lhs(cell_Wih.T)
    whh2h, whh2l = _split_lhs(cell_Whh.T)
    cbih2 = cell_bih.reshape(1, 3 * H)
    cbhh2 = cell_bhh.reshape(1, 3 * H)
    cbkh, cbkl = _split_lhs(codebook)
    cbkT = codebook.T
    cbTh, cbTl = _split_lhs(cbkT)
    wfh, wfl = _split_lhs(att_W.transpose(1, 0, 2).reshape(H, NHEADS * ATT_H))
    bf = att_b.reshape(1, NHEADS * ATT_H)
    uf = att_u.reshape(1, NHEADS * ATT_H)

    cnst = lambda i: (0, 0)
    tb = lambda i: (i, 0, 0)
    out = pl.pallas_call(
        _fused,
        grid=(NC + 1,),
        in_specs=[
            pl.BlockSpec((B, 1), cnst),
            pl.BlockSpec((TCH, B, CKP), tb),
            pl.BlockSpec((TCH, B, CKP), tb),
            pl.BlockSpec((TCH, B, CKP), tb),
            pl.BlockSpec((TCH, B, CKP), tb),
            pl.BlockSpec((TCH, B, CKP), tb),
            pl.BlockSpec((TCH, B, CKP), tb),
            pl.BlockSpec((CKP, D), cnst),
            pl.BlockSpec((CKP, D), cnst),
            pl.BlockSpec((CKP, D), cnst),
            pl.BlockSpec((CKP, D), cnst),
            pl.BlockSpec((CKP, D), cnst),
            pl.BlockSpec((CKP, D), cnst),
            pl.BlockSpec((1, D), cnst),
            pl.BlockSpec((D, 3 * H), cnst),
            pl.BlockSpec((D, 3 * H), cnst),
            pl.BlockSpec((1, 3 * H), cnst),
            pl.BlockSpec((H, 3 * H), cnst),
            pl.BlockSpec((H, 3 * H), cnst),
            pl.BlockSpec((1, 3 * H), cnst),
            pl.BlockSpec((NCODES, H), cnst),
            pl.BlockSpec((NCODES, H), cnst),
            pl.BlockSpec((H, NCODES), cnst),
            pl.BlockSpec((H, NCODES), cnst),
            pl.BlockSpec((H, NCODES), cnst),
            pl.BlockSpec((H, 3 * H), cnst),
            pl.BlockSpec((H, 3 * H), cnst),
            pl.BlockSpec((1, 3 * H), cnst),
            pl.BlockSpec((H, 3 * H), cnst),
            pl.BlockSpec((H, 3 * H), cnst),
            pl.BlockSpec((1, 3 * H), cnst),
            pl.BlockSpec((H, NHEADS * ATT_H), cnst),
            pl.BlockSpec((H, NHEADS * ATT_H), cnst),
            pl.BlockSpec((1, NHEADS * ATT_H), cnst),
            pl.BlockSpec((1, NHEADS * ATT_H), cnst),
        ],
        out_specs=pl.BlockSpec((B, NHEADS * H), cnst),
        out_shape=jax.ShapeDtypeStruct((B, NHEADS * H), F32),
        scratch_shapes=[
            pltpu.VMEM((B, H), F32),
            pltpu.VMEM((B, H), F32),
            pltpu.VMEM((8, B), jnp.int32),
            pltpu.VMEM((TCH, B, 3 * H), F32),
            pltpu.VMEM((TCH, B, 3 * H), F32),
            pltpu.VMEM((TCH, B, H), F32),
            pltpu.VMEM((TCH, B, H), F32),
            pltpu.VMEM((TCH, B, H), F32),
            pltpu.VMEM((TCH, B), jnp.int32),
            pltpu.VMEM((B, H), F32),
            pltpu.VMEM((B, H), F32),
            pltpu.VMEM((B, H), F32),
            pltpu.VMEM((B, H), F32),
            pltpu.VMEM((8, B), F32),
            pltpu.VMEM((8, B), F32),
            pltpu.VMEM((B, H), F32),
            pltpu.VMEM((8, NCODES), F32),
        ],
    )(l2, a0h, a0l, a1h, a1l, a2h, a2l, w0h, w0l, w1h, w1l, w2h, w2l, cb2,
      wihh, wihl, bih2, whhh, whhl, bhh2, cbkh, cbkl, cbkT, cbTh, cbTl,
      wih2h, wih2l, cbih2, whh2h, whh2l, cbhh2, wfh, wfl, bf, uf)

    return out


# speedup vs baseline: 1.8654x; 1.8654x over previous
import jax
import jax.numpy as jnp
from jax.experimental import pallas as pl
from jax.experimental.pallas import tpu as pltpu

B = 32
C = 39
L = 2048
D = 256
H = 256
K = 6
STRIDE = 2
NCODES = 1024
NHEADS = 4
ATT_H = 128

T = (L - K) // STRIDE + 1
TP = 1024
CK = C * STRIDE
CKP = 128

TCH = 64
NC = TP // TCH

F32 = jnp.float32
BF16 = jnp.bfloat16


def _split(a):
    ah = a.astype(BF16)
    al = (a - ah.astype(F32)).astype(BF16)
    return ah, al


def _mm(a, w):
    return jnp.dot(a.astype(BF16), w, preferred_element_type=F32)


def _gru_gates(g, gh, h):
    r = jax.nn.sigmoid(g[:, :H] + gh[:, :H])
    z = jax.nn.sigmoid(g[:, H:2 * H] + gh[:, H:2 * H])
    n = jnp.tanh(g[:, 2 * H:] + r * gh[:, 2 * H:])
    return (1.0 - z) * n + z * h


def _fused(l_ref, a0_ref, a1_ref, a2_ref, w0_ref, w1_ref, w2_ref, cb_ref,
           wih_ref, bih_ref, whh_ref, bhh_ref,
           cbkh_ref, cbkl_ref, cbkT_ref, cbT_ref,
           wih2_ref, bih2_ref, whh2_ref, bhh2_ref,
           wf_ref, bf_ref, uf_ref,
           out_ref,
           h1_ref, h2_ref, pidx_ref, gi1_ref, gi2_ref, y_ref, hs_ref,
           cont_ref, idx_ref, n0_ref, n1_ref, n2_ref, n3_ref, den_ref,
           scst_ref, hsst_ref, cn_ref):
    s = pl.program_id(0)
    nums = [n0_ref, n1_ref, n2_ref, n3_ref]

    @pl.when(s == 0)
    def _():
        h1_ref[...] = jnp.zeros_like(h1_ref)
        cbT = cbkT_ref[...]
        cn_ref[0:1] = jnp.sum(cbT * cbT, axis=0, keepdims=True)

    @pl.when(s == 1)
    def _():
        h2_ref[...] = jnp.zeros_like(h2_ref)
        pidx_ref[...] = jnp.full_like(pidx_ref, -1)
        for nr in nums:
            nr[...] = jnp.zeros_like(nr)
        den_ref[...] = jnp.zeros_like(den_ref)

    x = (jnp.dot(a0_ref[...].reshape(TCH * B, CKP), w0_ref[...],
                 preferred_element_type=F32)
         + jnp.dot(a1_ref[...].reshape(TCH * B, CKP), w1_ref[...],
                   preferred_element_type=F32)
         + jnp.dot(a2_ref[...].reshape(TCH * B, CKP), w2_ref[...],
                   preferred_element_type=F32)
         + cb_ref[...])
    gi1 = jnp.dot(x, wih_ref[...], preferred_element_type=F32) + bih_ref[...]
    gi1_ref[...] = gi1.reshape(TCH, B, 3 * H)

    @pl.when(s >= 1)
    def _():
        yf = y_ref[...].reshape(TCH * B, H)
        d = cn_ref[0:1] - 2.0 * jnp.dot(yf, cbT_ref[...], preferred_element_type=F32)
        d3 = d.reshape(TCH, B, NCODES)
        dmin = jnp.min(d3, axis=2, keepdims=True)
        iota = jax.lax.broadcasted_iota(jnp.int32, (TCH, B, NCODES), 2)
        cand = jnp.where(d3 == dmin, iota, NCODES)
        cmin = jnp.min(cand, axis=2, keepdims=True)
        idx3 = cmin[:, :, 0]
        idx_ref[...] = idx3
        onehot = (cand == cmin).astype(BF16).reshape(TCH * B, NCODES)
        q = (jnp.dot(onehot, cbkh_ref[...], preferred_element_type=F32)
             + jnp.dot(onehot, cbkl_ref[...], preferred_element_type=F32))
        gi2 = _mm(q, wih2_ref[...]) + bih2_ref[...]
        gi2_ref[...] = gi2.reshape(TCH, B, 3 * H)

        @pl.when(s >= 2)
        def _():
            nsd = (idx3[0:1] != pidx_ref[0:1]).astype(F32)
            hstash = hsst_ref[...]
            for k in range(NHEADS):
                wd = jnp.exp(scst_ref[k]) * nsd[0]
                nums[k][...] += wd[:, None] * hstash
                den_ref[k] += wd
            den_ref[NHEADS] += nsd[0]

        prev = jnp.concatenate([pidx_ref[0:1], idx3[:TCH - 1]], axis=0)
        cont3 = (idx3 == prev).astype(F32)[:, :, None]
        cont_ref[...] = jnp.broadcast_to(cont3, (TCH, B, H))
        pidx_ref[0:1] = idx3[TCH - 1:]

    whh1 = whh_ref[...]
    bhh1 = bhh_ref[...]
    whh2 = whh2_ref[...]
    bhh2 = bhh2_ref[...]

    def body(k, carry):
        h1, h2 = carry
        g1 = gi1_ref[pl.ds(k, 1)][0]
        gh1 = jnp.dot(h1, whh1, preferred_element_type=F32) + bhh1
        hn1 = _gru_gates(g1, gh1, h1)
        y_ref[pl.ds(k, 1)] = hn1[None]

        g2 = gi2_ref[pl.ds(k, 1)][0]
        c2 = cont_ref[pl.ds(k, 1)][0]
        hin = h2 * c2
        gh2 = _mm(hin, whh2) + bhh2
        hn2 = _gru_gates(g2, gh2, hin)
        hs_ref[pl.ds(k, 1)] = hn2[None]
        return hn1, hn2

    h1f, h2f = jax.lax.fori_loop(0, TCH, body,
                                 (h1_ref[...], h2_ref[...]))
    h1_ref[...] = h1f
    h2_ref[...] = h2f

    lc3 = ((l_ref[...] - (K - STRIDE)) // STRIDE)[None, :, :]
    tt3 = jax.lax.broadcasted_iota(jnp.int32, (TCH, B, 1), 0) + s * TCH
    y_ref[...] = y_ref[...] * (tt3 < lc3).astype(F32)

    @pl.when(s >= 1)
    def _():
        idx3 = idx_ref[...]
        hs3 = hs_ref[...]
        hsf = hs3.reshape(TCH * B, H)
        e = jnp.tanh(_mm(hsf, wf_ref[...]) + bf_ref[...])
        ew3 = (e * uf_ref[...]).reshape(TCH, B, NHEADS * ATT_H)
        tglob = (jax.lax.broadcasted_iota(jnp.int32, (TCH, B), 0)
                 + (s - 1) * TCH)
        nxt = jnp.concatenate([idx3[1:], idx3[TCH - 1:]], axis=0)
        nsr = jnp.logical_or(nxt != idx3, tglob == T - 1)
        m3 = jnp.logical_and(nsr, tglob < T).astype(F32)
        for k in range(NHEADS):
            sk = jnp.sum(ew3[:, :, k * ATT_H:(k + 1) * ATT_H], axis=2)
            wk = jnp.exp(sk) * m3
            nums[k][...] += jnp.sum(wk[:, :, None] * hs3, axis=0)
            den_ref[k] += jnp.sum(wk, axis=0)
            scst_ref[k:k + 1] = sk[TCH - 1:]
        den_ref[NHEADS] += jnp.sum(m3, axis=0)
        hsst_ref[...] = hs3[TCH - 1]

    @pl.when(s == NC)
    def _():
        ew0 = jnp.tanh(bf_ref[...]) * uf_ref[...]
        nseg = den_ref[NHEADS]
        ss = jnp.zeros((B, 1), F32)
        ps = []
        for k in range(NHEADS):
            s0 = jnp.sum(ew0[0, k * ATT_H:(k + 1) * ATT_H])
            dk = den_ref[k] + (float(T) - nseg) * jnp.exp(s0)
            pk = nums[k][...] / dk[:, None]
            ps.append(pk)
            ss = ss + jnp.sum(pk * pk, axis=1, keepdims=True)
        nrm = jnp.maximum(jnp.sqrt(ss), 1e-12)
        for k in range(NHEADS):
            out_ref[:, k * H:(k + 1) * H] = ps[k] / nrm


def kernel(input, l, conv_w, conv_b, gru_Wih, gru_Whh, gru_bih, gru_bhh,
           cell_Wih, cell_Whh, cell_bih, cell_bhh, codebook, att_W, att_b,
           att_u):
    p = input.reshape(B, C, L // STRIDE, STRIDE)
    p = p.transpose(2, 0, 1, 3).reshape(L // STRIDE, B, CK)
    p = jnp.pad(p, ((0, 0), (0, 0), (0, CKP - CK)))
    a0 = jnp.pad(p, ((0, TCH), (0, 0), (0, 0)))
    a1 = jnp.pad(p[1:], ((0, TCH + 1), (0, 0), (0, 0)))
    a2 = jnp.pad(p[2:], ((0, TCH + 2), (0, 0), (0, 0)))
    ws = []
    for j in range(K // STRIDE):
        wj = conv_w[:, :, STRIDE * j:STRIDE * (j + 1)].reshape(D, CK).T
        ws.append(jnp.pad(wj, ((0, CKP - CK), (0, 0))))
    w0, w1, w2 = ws
    l2 = l.astype(jnp.int32).reshape(B, 1)
    cb2 = conv_b.reshape(1, D)
    wihT = gru_Wih.T
    whhT = gru_Whh.T
    bih2 = gru_bih.reshape(1, 3 * H)
    bhh2 = gru_bhh.reshape(1, 3 * H)
    wih2T = cell_Wih.T.astype(BF16)
    whh2T = cell_Whh.T.astype(BF16)
    cbih2 = cell_bih.reshape(1, 3 * H)
    cbhh2 = cell_bhh.reshape(1, 3 * H)
    cbkh, cbkl = _split(codebook)
    cbkT = codebook.T
    cbTb = cbkT
    wf = att_W.transpose(1, 0, 2).reshape(H, NHEADS * ATT_H).astype(BF16)
    bf = att_b.reshape(1, NHEADS * ATT_H)
    uf = att_u.reshape(1, NHEADS * ATT_H)

    cnst = lambda i: (0, 0)
    tb = lambda i: (i, 0, 0)
    out = pl.pallas_call(
        _fused,
        grid=(NC + 1,),
        in_specs=[
            pl.BlockSpec((B, 1), cnst),
            pl.BlockSpec((TCH, B, CKP), tb),
            pl.BlockSpec((TCH, B, CKP), tb),
            pl.BlockSpec((TCH, B, CKP), tb),
            pl.BlockSpec((CKP, D), cnst),
            pl.BlockSpec((CKP, D), cnst),
            pl.BlockSpec((CKP, D), cnst),
            pl.BlockSpec((1, D), cnst),
            pl.BlockSpec((D, 3 * H), cnst),
            pl.BlockSpec((1, 3 * H), cnst),
            pl.BlockSpec((H, 3 * H), cnst),
            pl.BlockSpec((1, 3 * H), cnst),
            pl.BlockSpec((NCODES, H), cnst),
            pl.BlockSpec((NCODES, H), cnst),
            pl.BlockSpec((H, NCODES), cnst),
            pl.BlockSpec((H, NCODES), cnst),
            pl.BlockSpec((H, 3 * H), cnst),
            pl.BlockSpec((1, 3 * H), cnst),
            pl.BlockSpec((H, 3 * H), cnst),
            pl.BlockSpec((1, 3 * H), cnst),
            pl.BlockSpec((H, NHEADS * ATT_H), cnst),
            pl.BlockSpec((1, NHEADS * ATT_H), cnst),
            pl.BlockSpec((1, NHEADS * ATT_H), cnst),
        ],
        out_specs=pl.BlockSpec((B, NHEADS * H), cnst),
        out_shape=jax.ShapeDtypeStruct((B, NHEADS * H), F32),
        scratch_shapes=[
            pltpu.VMEM((B, H), F32),
            pltpu.VMEM((B, H), F32),
            pltpu.VMEM((8, B), jnp.int32),
            pltpu.VMEM((TCH, B, 3 * H), F32),
            pltpu.VMEM((TCH, B, 3 * H), F32),
            pltpu.VMEM((TCH, B, H), F32),
            pltpu.VMEM((TCH, B, H), F32),
            pltpu.VMEM((TCH, B, H), F32),
            pltpu.VMEM((TCH, B), jnp.int32),
            pltpu.VMEM((B, H), F32),
            pltpu.VMEM((B, H), F32),
            pltpu.VMEM((B, H), F32),
            pltpu.VMEM((B, H), F32),
            pltpu.VMEM((8, B), F32),
            pltpu.VMEM((8, B), F32),
            pltpu.VMEM((B, H), F32),
            pltpu.VMEM((8, NCODES), F32),
        ],
    )(l2, a0, a1, a2, w0, w1, w2, cb2, wihT, bih2, whhT, bhh2,
      cbkh, cbkl, cbkT, cbTb, wih2T, cbih2, whh2T, cbhh2, wf, bf, uf)

    return out


# speedup vs baseline: 1.9564x; 1.0488x over previous
import jax
import jax.numpy as jnp
from jax.experimental import pallas as pl
from jax.experimental.pallas import tpu as pltpu

B = 32
C = 39
L = 2048
D = 256
H = 256
K = 6
STRIDE = 2
NCODES = 1024
NHEADS = 4
ATT_H = 128

T = (L - K) // STRIDE + 1
TP = 1024
CK = C * STRIDE
CKP = 128

TCH = 64
NC = TP // TCH

F32 = jnp.float32


def _gru_gates(g, gh, h):
    r = jax.nn.sigmoid(g[:, :H] + gh[:, :H])
    z = jax.nn.sigmoid(g[:, H:2 * H] + gh[:, H:2 * H])
    n = jnp.tanh(g[:, 2 * H:] + r * gh[:, 2 * H:])
    return (1.0 - z) * n + z * h


def _fused(l_ref, a0_ref, a1_ref, a2_ref, w0_ref, w1_ref, w2_ref, cb_ref,
           wih_ref, bih_ref, whh_ref, bhh_ref,
           cbk_ref, cbkT_ref, wih2_ref, bih2_ref, whh2_ref, bhh2_ref,
           wf_ref, bf_ref, uf_ref,
           out_ref,
           h1_ref, h2_ref, pidx_ref, gi1_ref, gi2_ref, y_ref, hs_ref,
           cont_ref, idx_ref, n0_ref, n1_ref, n2_ref, n3_ref, den_ref,
           scst_ref, hsst_ref, cn_ref):
    s = pl.program_id(0)
    nums = [n0_ref, n1_ref, n2_ref, n3_ref]

    @pl.when(s == 0)
    def _():
        h1_ref[...] = jnp.zeros_like(h1_ref)
        cbT0 = cbkT_ref[...]
        cn_ref[0:1] = jnp.sum(cbT0 * cbT0, axis=0, keepdims=True)

    @pl.when(s == 1)
    def _():
        h2_ref[...] = jnp.zeros_like(h2_ref)
        pidx_ref[...] = jnp.full_like(pidx_ref, -1)
        for nr in nums:
            nr[...] = jnp.zeros_like(nr)
        den_ref[...] = jnp.zeros_like(den_ref)

    a0 = a0_ref[...].reshape(TCH * B, CKP)
    a1 = a1_ref[...].reshape(TCH * B, CKP)
    a2 = a2_ref[...].reshape(TCH * B, CKP)
    x = (jnp.dot(a0, w0_ref[...], preferred_element_type=F32)
         + jnp.dot(a1, w1_ref[...], preferred_element_type=F32)
         + jnp.dot(a2, w2_ref[...], preferred_element_type=F32)
         + cb_ref[...])
    gi1 = jnp.dot(x, wih_ref[...], preferred_element_type=F32) + bih_ref[...]
    gi1_ref[...] = gi1.reshape(TCH, B, 3 * H)

    @pl.when(s >= 1)
    def _():
        yf = y_ref[...].reshape(TCH * B, H)
        d = (cn_ref[0:1]
             - 2.0 * jnp.dot(yf, cbkT_ref[...], preferred_element_type=F32))
        d3 = d.reshape(TCH, B, NCODES)
        dmin = jnp.min(d3, axis=2, keepdims=True)
        iota = jax.lax.broadcasted_iota(jnp.int32, (TCH, B, NCODES), 2)
        cand = jnp.where(d3 == dmin, iota, NCODES)
        cmin = jnp.min(cand, axis=2, keepdims=True)
        idx3 = cmin[:, :, 0]
        idx_ref[...] = idx3
        onehot = (cand == cmin).astype(F32).reshape(TCH * B, NCODES)
        q = jnp.dot(onehot, cbk_ref[...], preferred_element_type=F32)
        gi2 = (jnp.dot(q, wih2_ref[...], preferred_element_type=F32)
               + bih2_ref[...])
        gi2_ref[...] = gi2.reshape(TCH, B, 3 * H)

        @pl.when(s >= 2)
        def _():
            nsd = (idx3[0:1] != pidx_ref[0:1]).astype(F32)
            hstash = hsst_ref[...]
            for k in range(NHEADS):
                wd = jnp.exp(scst_ref[k]) * nsd[0]
                nums[k][...] += wd[:, None] * hstash
                den_ref[k] += wd
            den_ref[NHEADS] += nsd[0]

        prev = jnp.concatenate([pidx_ref[0:1], idx3[:TCH - 1]], axis=0)
        cont3 = (idx3 == prev).astype(F32)[:, :, None]
        cont_ref[...] = jnp.broadcast_to(cont3, (TCH, B, H))
        pidx_ref[0:1] = idx3[TCH - 1:]

    whh1 = whh_ref[...]
    bhh1 = bhh_ref[...]
    whh2 = whh2_ref[...]
    bhh2 = bhh2_ref[...]

    def body(k, carry):
        h1, h2 = carry
        g1 = gi1_ref[pl.ds(k, 1)][0]
        gh1 = jnp.dot(h1, whh1, preferred_element_type=F32) + bhh1
        hn1 = _gru_gates(g1, gh1, h1)
        y_ref[pl.ds(k, 1)] = hn1[None]

        g2 = gi2_ref[pl.ds(k, 1)][0]
        c2 = cont_ref[pl.ds(k, 1)][0]
        hin = h2 * c2
        gh2 = jnp.dot(hin, whh2, preferred_element_type=F32) + bhh2
        hn2 = _gru_gates(g2, gh2, hin)
        hs_ref[pl.ds(k, 1)] = hn2[None]
        return hn1, hn2

    h1f, h2f = jax.lax.fori_loop(0, TCH, body,
                                 (h1_ref[...], h2_ref[...]))
    h1_ref[...] = h1f
    h2_ref[...] = h2f

    lc3 = ((l_ref[...] - (K - STRIDE)) // STRIDE)[None, :, :]
    tt3 = jax.lax.broadcasted_iota(jnp.int32, (TCH, B, 1), 0) + s * TCH
    y_ref[...] = y_ref[...] * (tt3 < lc3).astype(F32)

    @pl.when(s >= 1)
    def _():
        idx3 = idx_ref[...]
        hs3 = hs_ref[...]
        hsf = hs3.reshape(TCH * B, H)
        e = jnp.tanh(jnp.dot(hsf, wf_ref[...], preferred_element_type=F32)
                     + bf_ref[...])
        ew3 = (e * uf_ref[...]).reshape(TCH, B, NHEADS * ATT_H)
        tglob = (jax.lax.broadcasted_iota(jnp.int32, (TCH, B), 0)
                 + (s - 1) * TCH)
        nxt = jnp.concatenate([idx3[1:], idx3[TCH - 1:]], axis=0)
        nsr = jnp.logical_or(nxt != idx3, tglob == T - 1)
        m3 = jnp.logical_and(nsr, tglob < T).astype(F32)
        for k in range(NHEADS):
            sk = jnp.sum(ew3[:, :, k * ATT_H:(k + 1) * ATT_H], axis=2)
            wk = jnp.exp(sk) * m3
            nums[k][...] += jnp.sum(wk[:, :, None] * hs3, axis=0)
            den_ref[k] += jnp.sum(wk, axis=0)
            scst_ref[k:k + 1] = sk[TCH - 1:]
        den_ref[NHEADS] += jnp.sum(m3, axis=0)
        hsst_ref[...] = hs3[TCH - 1]

    @pl.when(s == NC)
    def _():
        ew0 = jnp.tanh(bf_ref[...]) * uf_ref[...]
        nseg = den_ref[NHEADS]
        ss = jnp.zeros((B, 1), F32)
        ps = []
        for k in range(NHEADS):
            s0 = jnp.sum(ew0[0, k * ATT_H:(k + 1) * ATT_H])
            dk = den_ref[k] + (float(T) - nseg) * jnp.exp(s0)
            pk = nums[k][...] / dk[:, None]
            ps.append(pk)
            ss = ss + jnp.sum(pk * pk, axis=1, keepdims=True)
        nrm = jnp.maximum(jnp.sqrt(ss), 1e-12)
        for k in range(NHEADS):
            out_ref[:, k * H:(k + 1) * H] = ps[k] / nrm


def kernel(input, l, conv_w, conv_b, gru_Wih, gru_Whh, gru_bih, gru_bhh,
           cell_Wih, cell_Whh, cell_bih, cell_bhh, codebook, att_W, att_b,
           att_u):
    p = input.reshape(B, C, L // STRIDE, STRIDE)
    p = p.transpose(2, 0, 1, 3).reshape(L // STRIDE, B, CK)
    p = jnp.pad(p, ((0, 0), (0, 0), (0, CKP - CK)))
    a0 = jnp.pad(p, ((0, TCH), (0, 0), (0, 0)))
    a1 = jnp.pad(p[1:], ((0, TCH + 1), (0, 0), (0, 0)))
    a2 = jnp.pad(p[2:], ((0, TCH + 2), (0, 0), (0, 0)))
    ws = []
    for j in range(K // STRIDE):
        wj = conv_w[:, :, STRIDE * j:STRIDE * (j + 1)].reshape(D, CK).T
        ws.append(jnp.pad(wj, ((0, CKP - CK), (0, 0))))
    w0, w1, w2 = ws
    l2 = l.astype(jnp.int32).reshape(B, 1)
    cb2 = conv_b.reshape(1, D)
    wihT = gru_Wih.T
    whhT = gru_Whh.T
    bih2 = gru_bih.reshape(1, 3 * H)
    bhh2 = gru_bhh.reshape(1, 3 * H)
    wih2T = cell_Wih.T
    whh2T = cell_Whh.T
    cbih2 = cell_bih.reshape(1, 3 * H)
    cbhh2 = cell_bhh.reshape(1, 3 * H)
    cbkT = codebook.T
    wf = att_W.transpose(1, 0, 2).reshape(H, NHEADS * ATT_H)
    bf = att_b.reshape(1, NHEADS * ATT_H)
    uf = att_u.reshape(1, NHEADS * ATT_H)

    cnst = lambda i: (0, 0)
    out = pl.pallas_call(
        _fused,
        grid=(NC + 1,),
        in_specs=[
            pl.BlockSpec((B, 1), cnst),
            pl.BlockSpec((TCH, B, CKP), lambda i: (i, 0, 0)),
            pl.BlockSpec((TCH, B, CKP), lambda i: (i, 0, 0)),
            pl.BlockSpec((TCH, B, CKP), lambda i: (i, 0, 0)),
            pl.BlockSpec((CKP, D), cnst),
            pl.BlockSpec((CKP, D), cnst),
            pl.BlockSpec((CKP, D), cnst),
            pl.BlockSpec((1, D), cnst),
            pl.BlockSpec((D, 3 * H), cnst),
            pl.BlockSpec((1, 3 * H), cnst),
            pl.BlockSpec((H, 3 * H), cnst),
            pl.BlockSpec((1, 3 * H), cnst),
            pl.BlockSpec((NCODES, H), cnst),
            pl.BlockSpec((H, NCODES), cnst),
            pl.BlockSpec((H, 3 * H), cnst),
            pl.BlockSpec((1, 3 * H), cnst),
            pl.BlockSpec((H, 3 * H), cnst),
            pl.BlockSpec((1, 3 * H), cnst),
            pl.BlockSpec((H, NHEADS * ATT_H), cnst),
            pl.BlockSpec((1, NHEADS * ATT_H), cnst),
            pl.BlockSpec((1, NHEADS * ATT_H), cnst),
        ],
        out_specs=pl.BlockSpec((B, NHEADS * H), cnst),
        out_shape=jax.ShapeDtypeStruct((B, NHEADS * H), F32),
        scratch_shapes=[
            pltpu.VMEM((B, H), F32),
            pltpu.VMEM((B, H), F32),
            pltpu.VMEM((8, B), jnp.int32),
            pltpu.VMEM((TCH, B, 3 * H), F32),
            pltpu.VMEM((TCH, B, 3 * H), F32),
            pltpu.VMEM((TCH, B, H), F32),
            pltpu.VMEM((TCH, B, H), F32),
            pltpu.VMEM((TCH, B, H), F32),
            pltpu.VMEM((TCH, B), jnp.int32),
            pltpu.VMEM((B, H), F32),
            pltpu.VMEM((B, H), F32),
            pltpu.VMEM((B, H), F32),
            pltpu.VMEM((B, H), F32),
            pltpu.VMEM((8, B), F32),
            pltpu.VMEM((8, B), F32),
            pltpu.VMEM((B, H), F32),
            pltpu.VMEM((8, NCODES), F32),
        ],
    )(l2, a0, a1, a2, w0, w1, w2, cb2, wihT, bih2, whhT, bhh2,
      codebook, cbkT, wih2T, cbih2, whh2T, cbhh2, wf, bf, uf)

    return out


# speedup vs baseline: 2.1536x; 1.1008x over previous
import jax
import jax.numpy as jnp
from jax.experimental import pallas as pl
from jax.experimental.pallas import tpu as pltpu

B = 32
C = 39
L = 2048
D = 256
H = 256
K = 6
STRIDE = 2
NCODES = 1024
NHEADS = 4
ATT_H = 128

T = (L - K) // STRIDE + 1
TP = 1024
CK = C * STRIDE
CKP = 128

TCH = 64
NC = TP // TCH

F32 = jnp.float32


def _gru_gates(g, gh, h):
    r = jax.nn.sigmoid(g[:, :H] + gh[:, :H])
    z = jax.nn.sigmoid(g[:, H:2 * H] + gh[:, H:2 * H])
    n = jnp.tanh(g[:, 2 * H:] + r * gh[:, 2 * H:])
    return (1.0 - z) * n + z * h


def _fused(l_ref, a0_ref, a1_ref, a2_ref, w0_ref, w1_ref, w2_ref, cb_ref,
           wih_ref, bih_ref, whh_ref, bhh_ref,
           cbk_ref, cbkT_ref, wih2_ref, bih2_ref, whh2_ref, bhh2_ref,
           wf_ref, bf_ref, uf_ref,
           out_ref,
           h1_ref, h2_ref, pidx_ref, gi1_ref, gi2_ref, y_ref, hs_ref,
           cont_ref, idx_ref, n0_ref, n1_ref, n2_ref, n3_ref, den_ref,
           scst_ref, hsst_ref, cn_ref):
    s = pl.program_id(0)
    nums = [n0_ref, n1_ref, n2_ref, n3_ref]

    @pl.when(s == 0)
    def _():
        h1_ref[...] = jnp.zeros_like(h1_ref)
        cbT0 = cbkT_ref[...]
        cn_ref[0:1] = jnp.sum(cbT0 * cbT0, axis=0, keepdims=True)

    @pl.when(s == 1)
    def _():
        h2_ref[...] = jnp.zeros_like(h2_ref)
        pidx_ref[...] = jnp.full_like(pidx_ref, -1)
        for nr in nums:
            nr[...] = jnp.zeros_like(nr)
        den_ref[...] = jnp.zeros_like(den_ref)

    a0 = a0_ref[...].reshape(TCH * B, CKP)
    a1 = a1_ref[...].reshape(TCH * B, CKP)
    a2 = a2_ref[...].reshape(TCH * B, CKP)
    x = (jnp.dot(a0, w0_ref[...], preferred_element_type=F32)
         + jnp.dot(a1, w1_ref[...], preferred_element_type=F32)
         + jnp.dot(a2, w2_ref[...], preferred_element_type=F32)
         + cb_ref[...])
    gi1 = jnp.dot(x, wih_ref[...], preferred_element_type=F32) + bih_ref[...]
    gi1_ref[...] = gi1.reshape(TCH, B, 3 * H)

    @pl.when(s >= 1)
    def _():
        yf = y_ref[...].reshape(TCH * B, H)
        d = (cn_ref[0:1]
             - 2.0 * jnp.dot(yf, cbkT_ref[...], preferred_element_type=F32))
        d3 = d.reshape(TCH, B, NCODES)
        dmin = jnp.min(d3, axis=2, keepdims=True)
        iota = jax.lax.broadcasted_iota(jnp.int32, (TCH, B, NCODES), 2)
        cand = jnp.where(d3 == dmin, iota, NCODES)
        cmin = jnp.min(cand, axis=2, keepdims=True)
        idx3 = cmin[:, :, 0]
        idx_ref[...] = idx3
        onehot = (cand == cmin).astype(F32).reshape(TCH * B, NCODES)
        q = jnp.dot(onehot, cbk_ref[...], preferred_element_type=F32)
        gi2 = (jnp.dot(q, wih2_ref[...], preferred_element_type=F32)
               + bih2_ref[...])
        gi2_ref[...] = gi2.reshape(TCH, B, 3 * H)

        @pl.when(s >= 2)
        def _():
            nsd = (idx3[0:1] != pidx_ref[0:1]).astype(F32)
            hstash = hsst_ref[...]
            for k in range(NHEADS):
                wd = jnp.exp(scst_ref[k]) * nsd[0]
                nums[k][...] += wd[:, None] * hstash
                den_ref[k] += wd
            den_ref[NHEADS] += nsd[0]

        prev = jnp.concatenate([pidx_ref[0:1], idx3[:TCH - 1]], axis=0)
        cont3 = (idx3 == prev).astype(F32)[:, :, None]
        cont_ref[...] = jnp.broadcast_to(cont3, (TCH, B, H))
        pidx_ref[0:1] = idx3[TCH - 1:]

    whh1 = whh_ref[...]
    bhh1 = bhh_ref[...]
    whh2 = whh2_ref[...]
    bhh2 = bhh2_ref[...]

    def body(k, carry):
        h1, h2 = carry
        g1 = gi1_ref[pl.ds(k, 1)][0]
        gh1 = jnp.dot(h1, whh1, preferred_element_type=F32) + bhh1
        hn1 = _gru_gates(g1, gh1, h1)
        y_ref[pl.ds(k, 1)] = hn1[None]

        g2 = gi2_ref[pl.ds(k, 1)][0]
        c2 = cont_ref[pl.ds(k, 1)][0]
        hin = h2 * c2
        gh2 = jnp.dot(hin, whh2, preferred_element_type=F32) + bhh2
        hn2 = _gru_gates(g2, gh2, hin)
        hs_ref[pl.ds(k, 1)] = hn2[None]
        return hn1, hn2

    h1f, h2f = jax.lax.fori_loop(0, TCH, body,
                                 (h1_ref[...], h2_ref[...]), unroll=2)
    h1_ref[...] = h1f
    h2_ref[...] = h2f

    lc3 = ((l_ref[...] - (K - STRIDE)) // STRIDE)[None, :, :]
    tt3 = jax.lax.broadcasted_iota(jnp.int32, (TCH, B, 1), 0) + s * TCH
    y_ref[...] = y_ref[...] * (tt3 < lc3).astype(F32)

    @pl.when(s >= 1)
    def _():
        idx3 = idx_ref[...]
        hs3 = hs_ref[...]
        hsf = hs3.reshape(TCH * B, H)
        e = jnp.tanh(jnp.dot(hsf, wf_ref[...], preferred_element_type=F32)
                     + bf_ref[...])
        ew3 = (e * uf_ref[...]).reshape(TCH, B, NHEADS * ATT_H)
        tglob = (jax.lax.broadcasted_iota(jnp.int32, (TCH, B), 0)
                 + (s - 1) * TCH)
        nxt = jnp.concatenate([idx3[1:], idx3[TCH - 1:]], axis=0)
        nsr = jnp.logical_or(nxt != idx3, tglob == T - 1)
        m3 = jnp.logical_and(nsr, tglob < T).astype(F32)
        for k in range(NHEADS):
            sk = jnp.sum(ew3[:, :, k * ATT_H:(k + 1) * ATT_H], axis=2)
            wk = jnp.exp(sk) * m3
            nums[k][...] += jnp.sum(wk[:, :, None] * hs3, axis=0)
            den_ref[k] += jnp.sum(wk, axis=0)
            scst_ref[k:k + 1] = sk[TCH - 1:]
        den_ref[NHEADS] += jnp.sum(m3, axis=0)
        hsst_ref[...] = hs3[TCH - 1]

    @pl.when(s == NC)
    def _():
        ew0 = jnp.tanh(bf_ref[...]) * uf_ref[...]
        nseg = den_ref[NHEADS]
        ss = jnp.zeros((B, 1), F32)
        ps = []
        for k in range(NHEADS):
            s0 = jnp.sum(ew0[0, k * ATT_H:(k + 1) * ATT_H])
            dk = den_ref[k] + (float(T) - nseg) * jnp.exp(s0)
            pk = nums[k][...] / dk[:, None]
            ps.append(pk)
            ss = ss + jnp.sum(pk * pk, axis=1, keepdims=True)
        nrm = jnp.maximum(jnp.sqrt(ss), 1e-12)
        for k in range(NHEADS):
            out_ref[:, k * H:(k + 1) * H] = ps[k] / nrm


def kernel(input, l, conv_w, conv_b, gru_Wih, gru_Whh, gru_bih, gru_bhh,
           cell_Wih, cell_Whh, cell_bih, cell_bhh, codebook, att_W, att_b,
           att_u):
    p = input.reshape(B, C, L // STRIDE, STRIDE)
    p = p.transpose(2, 0, 1, 3).reshape(L // STRIDE, B, CK)
    p = jnp.pad(p, ((0, 0), (0, 0), (0, CKP - CK)))
    a0 = jnp.pad(p, ((0, TCH), (0, 0), (0, 0)))
    a1 = jnp.pad(p[1:], ((0, TCH + 1), (0, 0), (0, 0)))
    a2 = jnp.pad(p[2:], ((0, TCH + 2), (0, 0), (0, 0)))
    ws = []
    for j in range(K // STRIDE):
        wj = conv_w[:, :, STRIDE * j:STRIDE * (j + 1)].reshape(D, CK).T
        ws.append(jnp.pad(wj, ((0, CKP - CK), (0, 0))))
    w0, w1, w2 = ws
    l2 = l.astype(jnp.int32).reshape(B, 1)
    cb2 = conv_b.reshape(1, D)
    wihT = gru_Wih.T
    whhT = gru_Whh.T
    bih2 = gru_bih.reshape(1, 3 * H)
    bhh2 = gru_bhh.reshape(1, 3 * H)
    wih2T = cell_Wih.T
    whh2T = cell_Whh.T
    cbih2 = cell_bih.reshape(1, 3 * H)
    cbhh2 = cell_bhh.reshape(1, 3 * H)
    cbkT = codebook.T
    wf = att_W.transpose(1, 0, 2).reshape(H, NHEADS * ATT_H)
    bf = att_b.reshape(1, NHEADS * ATT_H)
    uf = att_u.reshape(1, NHEADS * ATT_H)

    cnst = lambda i: (0, 0)
    out = pl.pallas_call(
        _fused,
        grid=(NC + 1,),
        in_specs=[
            pl.BlockSpec((B, 1), cnst),
            pl.BlockSpec((TCH, B, CKP), lambda i: (i, 0, 0)),
            pl.BlockSpec((TCH, B, CKP), lambda i: (i, 0, 0)),
            pl.BlockSpec((TCH, B, CKP), lambda i: (i, 0, 0)),
            pl.BlockSpec((CKP, D), cnst),
            pl.BlockSpec((CKP, D), cnst),
            pl.BlockSpec((CKP, D), cnst),
            pl.BlockSpec((1, D), cnst),
            pl.BlockSpec((D, 3 * H), cnst),
            pl.BlockSpec((1, 3 * H), cnst),
            pl.BlockSpec((H, 3 * H), cnst),
            pl.BlockSpec((1, 3 * H), cnst),
            pl.BlockSpec((NCODES, H), cnst),
            pl.BlockSpec((H, NCODES), cnst),
            pl.BlockSpec((H, 3 * H), cnst),
            pl.BlockSpec((1, 3 * H), cnst),
            pl.BlockSpec((H, 3 * H), cnst),
            pl.BlockSpec((1, 3 * H), cnst),
            pl.BlockSpec((H, NHEADS * ATT_H), cnst),
            pl.BlockSpec((1, NHEADS * ATT_H), cnst),
            pl.BlockSpec((1, NHEADS * ATT_H), cnst),
        ],
        out_specs=pl.BlockSpec((B, NHEADS * H), cnst),
        out_shape=jax.ShapeDtypeStruct((B, NHEADS * H), F32),
        scratch_shapes=[
            pltpu.VMEM((B, H), F32),
            pltpu.VMEM((B, H), F32),
            pltpu.VMEM((8, B), jnp.int32),
            pltpu.VMEM((TCH, B, 3 * H), F32),
            pltpu.VMEM((TCH, B, 3 * H), F32),
            pltpu.VMEM((TCH, B, H), F32),
            pltpu.VMEM((TCH, B, H), F32),
            pltpu.VMEM((TCH, B, H), F32),
            pltpu.VMEM((TCH, B), jnp.int32),
            pltpu.VMEM((B, H), F32),
            pltpu.VMEM((B, H), F32),
            pltpu.VMEM((B, H), F32),
            pltpu.VMEM((B, H), F32),
            pltpu.VMEM((8, B), F32),
            pltpu.VMEM((8, B), F32),
            pltpu.VMEM((B, H), F32),
            pltpu.VMEM((8, NCODES), F32),
        ],
    )(l2, a0, a1, a2, w0, w1, w2, cb2, wihT, bih2, whhT, bhh2,
      codebook, cbkT, wih2T, cbih2, whh2T, cbhh2, wf, bf, uf)

    return out


# speedup vs baseline: 2.2508x; 1.0452x over previous
import jax
import jax.numpy as jnp
from jax.experimental import pallas as pl
from jax.experimental.pallas import tpu as pltpu

B = 32
C = 39
L = 2048
D = 256
H = 256
K = 6
STRIDE = 2
NCODES = 1024
NHEADS = 4
ATT_H = 128

T = (L - K) // STRIDE + 1
TP = 1024
CK = C * STRIDE
CKP = 128

TCH = 64
NC = TP // TCH

F32 = jnp.float32


def _gru_gates(g, gh, h):
    r = jax.nn.sigmoid(g[:, :H] + gh[:, :H])
    z = jax.nn.sigmoid(g[:, H:2 * H] + gh[:, H:2 * H])
    n = jnp.tanh(g[:, 2 * H:] + r * gh[:, 2 * H:])
    return (1.0 - z) * n + z * h


def _fused(l_ref, a0_ref, a1_ref, a2_ref, w0_ref, w1_ref, w2_ref, cb_ref,
           wih_ref, bih_ref, whh_ref, bhh_ref,
           cbk_ref, cbkT_ref, wih2_ref, bih2_ref, whh2_ref, bhh2_ref,
           wf_ref, bf_ref, uf_ref,
           out_ref,
           h1_ref, h2_ref, pidx_ref, gi1_ref, gi2_ref, y_ref, hs_ref,
           cont_ref, idx_ref, n0_ref, n1_ref, n2_ref, n3_ref, den_ref,
           scst_ref, hsst_ref, cn_ref):
    s = pl.program_id(0)
    nums = [n0_ref, n1_ref, n2_ref, n3_ref]

    @pl.when(s == 0)
    def _():
        h1_ref[...] = jnp.zeros_like(h1_ref)
        cbT0 = cbkT_ref[...]
        cn_ref[0:1] = jnp.sum(cbT0 * cbT0, axis=0, keepdims=True)

    @pl.when(s == 1)
    def _():
        h2_ref[...] = jnp.zeros_like(h2_ref)
        pidx_ref[...] = jnp.full_like(pidx_ref, -1)
        for nr in nums:
            nr[...] = jnp.zeros_like(nr)
        den_ref[...] = jnp.zeros_like(den_ref)

    a0 = a0_ref[...].reshape(TCH * B, CKP)
    a1 = a1_ref[...].reshape(TCH * B, CKP)
    a2 = a2_ref[...].reshape(TCH * B, CKP)
    x = (jnp.dot(a0, w0_ref[...], preferred_element_type=F32)
         + jnp.dot(a1, w1_ref[...], preferred_element_type=F32)
         + jnp.dot(a2, w2_ref[...], preferred_element_type=F32)
         + cb_ref[...])
    gi1 = jnp.dot(x, wih_ref[...], preferred_element_type=F32) + bih_ref[...]
    gi1_ref[...] = gi1.reshape(TCH, B, 3 * H)

    @pl.when(s >= 1)
    def _():
        yf = y_ref[...].reshape(TCH * B, H)
        d = (cn_ref[0:1]
             - 2.0 * jnp.dot(yf, cbkT_ref[...], preferred_element_type=F32))
        d3 = d.reshape(TCH, B, NCODES)
        dmin = jnp.min(d3, axis=2, keepdims=True)
        iota = jax.lax.broadcasted_iota(jnp.int32, (TCH, B, NCODES), 2)
        cand = jnp.where(d3 == dmin, iota, NCODES)
        cmin = jnp.min(cand, axis=2, keepdims=True)
        idx3 = cmin[:, :, 0]
        idx_ref[...] = idx3
        onehot = (cand == cmin).astype(F32).reshape(TCH * B, NCODES)
        q = jnp.dot(onehot, cbk_ref[...], preferred_element_type=F32)
        gi2 = (jnp.dot(q, wih2_ref[...], preferred_element_type=F32)
               + bih2_ref[...])
        gi2_ref[...] = gi2.reshape(TCH, B, 3 * H)

        @pl.when(s >= 2)
        def _():
            nsd = (idx3[0:1] != pidx_ref[0:1]).astype(F32)
            hstash = hsst_ref[...]
            for k in range(NHEADS):
                wd = jnp.exp(scst_ref[k]) * nsd[0]
                nums[k][...] += wd[:, None] * hstash
                den_ref[k] += wd
            den_ref[NHEADS] += nsd[0]

        prev = jnp.concatenate([pidx_ref[0:1], idx3[:TCH - 1]], axis=0)
        cont3 = (idx3 == prev).astype(F32)[:, :, None]
        cont_ref[...] = jnp.broadcast_to(cont3, (TCH, B, H))
        pidx_ref[0:1] = idx3[TCH - 1:]

    whh1 = whh_ref[...]
    bhh1 = bhh_ref[...]
    whh2 = whh2_ref[...]
    bhh2 = bhh2_ref[...]

    def body(k, carry):
        h1, h2 = carry
        g1 = gi1_ref[pl.ds(k, 1)][0]
        gh1 = jnp.dot(h1, whh1, preferred_element_type=F32) + bhh1
        hn1 = _gru_gates(g1, gh1, h1)
        y_ref[pl.ds(k, 1)] = hn1[None]

        g2 = gi2_ref[pl.ds(k, 1)][0]
        c2 = cont_ref[pl.ds(k, 1)][0]
        hin = h2 * c2
        gh2 = jnp.dot(hin, whh2, preferred_element_type=F32) + bhh2
        hn2 = _gru_gates(g2, gh2, hin)
        hs_ref[pl.ds(k, 1)] = hn2[None]
        return hn1, hn2

    h1f, h2f = jax.lax.fori_loop(0, TCH, body,
                                 (h1_ref[...], h2_ref[...]), unroll=4)
    h1_ref[...] = h1f
    h2_ref[...] = h2f

    lc3 = ((l_ref[...] - (K - STRIDE)) // STRIDE)[None, :, :]
    tt3 = jax.lax.broadcasted_iota(jnp.int32, (TCH, B, 1), 0) + s * TCH
    y_ref[...] = y_ref[...] * (tt3 < lc3).astype(F32)

    @pl.when(s >= 1)
    def _():
        idx3 = idx_ref[...]
        hs3 = hs_ref[...]
        hsf = hs3.reshape(TCH * B, H)
        e = jnp.tanh(jnp.dot(hsf, wf_ref[...], preferred_element_type=F32)
                     + bf_ref[...])
        ew3 = (e * uf_ref[...]).reshape(TCH, B, NHEADS * ATT_H)
        tglob = (jax.lax.broadcasted_iota(jnp.int32, (TCH, B), 0)
                 + (s - 1) * TCH)
        nxt = jnp.concatenate([idx3[1:], idx3[TCH - 1:]], axis=0)
        nsr = jnp.logical_or(nxt != idx3, tglob == T - 1)
        m3 = jnp.logical_and(nsr, tglob < T).astype(F32)
        for k in range(NHEADS):
            sk = jnp.sum(ew3[:, :, k * ATT_H:(k + 1) * ATT_H], axis=2)
            wk = jnp.exp(sk) * m3
            nums[k][...] += jnp.sum(wk[:, :, None] * hs3, axis=0)
            den_ref[k] += jnp.sum(wk, axis=0)
            scst_ref[k:k + 1] = sk[TCH - 1:]
        den_ref[NHEADS] += jnp.sum(m3, axis=0)
        hsst_ref[...] = hs3[TCH - 1]

    @pl.when(s == NC)
    def _():
        ew0 = jnp.tanh(bf_ref[...]) * uf_ref[...]
        nseg = den_ref[NHEADS]
        ss = jnp.zeros((B, 1), F32)
        ps = []
        for k in range(NHEADS):
            s0 = jnp.sum(ew0[0, k * ATT_H:(k + 1) * ATT_H])
            dk = den_ref[k] + (float(T) - nseg) * jnp.exp(s0)
            pk = nums[k][...] / dk[:, None]
            ps.append(pk)
            ss = ss + jnp.sum(pk * pk, axis=1, keepdims=True)
        nrm = jnp.maximum(jnp.sqrt(ss), 1e-12)
        for k in range(NHEADS):
            out_ref[:, k * H:(k + 1) * H] = ps[k] / nrm


def kernel(input, l, conv_w, conv_b, gru_Wih, gru_Whh, gru_bih, gru_bhh,
           cell_Wih, cell_Whh, cell_bih, cell_bhh, codebook, att_W, att_b,
           att_u):
    p = input.reshape(B, C, L // STRIDE, STRIDE)
    p = p.transpose(2, 0, 1, 3).reshape(L // STRIDE, B, CK)
    p = jnp.pad(p, ((0, 0), (0, 0), (0, CKP - CK)))
    a0 = jnp.pad(p, ((0, TCH), (0, 0), (0, 0)))
    a1 = jnp.pad(p[1:], ((0, TCH + 1), (0, 0), (0, 0)))
    a2 = jnp.pad(p[2:], ((0, TCH + 2), (0, 0), (0, 0)))
    ws = []
    for j in range(K // STRIDE):
        wj = conv_w[:, :, STRIDE * j:STRIDE * (j + 1)].reshape(D, CK).T
        ws.append(jnp.pad(wj, ((0, CKP - CK), (0, 0))))
    w0, w1, w2 = ws
    l2 = l.astype(jnp.int32).reshape(B, 1)
    cb2 = conv_b.reshape(1, D)
    wihT = gru_Wih.T
    whhT = gru_Whh.T
    bih2 = gru_bih.reshape(1, 3 * H)
    bhh2 = gru_bhh.reshape(1, 3 * H)
    wih2T = cell_Wih.T
    whh2T = cell_Whh.T
    cbih2 = cell_bih.reshape(1, 3 * H)
    cbhh2 = cell_bhh.reshape(1, 3 * H)
    cbkT = codebook.T
    wf = att_W.transpose(1, 0, 2).reshape(H, NHEADS * ATT_H)
    bf = att_b.reshape(1, NHEADS * ATT_H)
    uf = att_u.reshape(1, NHEADS * ATT_H)

    cnst = lambda i: (0, 0)
    out = pl.pallas_call(
        _fused,
        grid=(NC + 1,),
        in_specs=[
            pl.BlockSpec((B, 1), cnst),
            pl.BlockSpec((TCH, B, CKP), lambda i: (i, 0, 0)),
            pl.BlockSpec((TCH, B, CKP), lambda i: (i, 0, 0)),
            pl.BlockSpec((TCH, B, CKP), lambda i: (i, 0, 0)),
            pl.BlockSpec((CKP, D), cnst),
            pl.BlockSpec((CKP, D), cnst),
            pl.BlockSpec((CKP, D), cnst),
            pl.BlockSpec((1, D), cnst),
            pl.BlockSpec((D, 3 * H), cnst),
            pl.BlockSpec((1, 3 * H), cnst),
            pl.BlockSpec((H, 3 * H), cnst),
            pl.BlockSpec((1, 3 * H), cnst),
            pl.BlockSpec((NCODES, H), cnst),
            pl.BlockSpec((H, NCODES), cnst),
            pl.BlockSpec((H, 3 * H), cnst),
            pl.BlockSpec((1, 3 * H), cnst),
            pl.BlockSpec((H, 3 * H), cnst),
            pl.BlockSpec((1, 3 * H), cnst),
            pl.BlockSpec((H, NHEADS * ATT_H), cnst),
            pl.BlockSpec((1, NHEADS * ATT_H), cnst),
            pl.BlockSpec((1, NHEADS * ATT_H), cnst),
        ],
        out_specs=pl.BlockSpec((B, NHEADS * H), cnst),
        out_shape=jax.ShapeDtypeStruct((B, NHEADS * H), F32),
        scratch_shapes=[
            pltpu.VMEM((B, H), F32),
            pltpu.VMEM((B, H), F32),
            pltpu.VMEM((8, B), jnp.int32),
            pltpu.VMEM((TCH, B, 3 * H), F32),
            pltpu.VMEM((TCH, B, 3 * H), F32),
            pltpu.VMEM((TCH, B, H), F32),
            pltpu.VMEM((TCH, B, H), F32),
            pltpu.VMEM((TCH, B, H), F32),
            pltpu.VMEM((TCH, B), jnp.int32),
            pltpu.VMEM((B, H), F32),
            pltpu.VMEM((B, H), F32),
            pltpu.VMEM((B, H), F32),
            pltpu.VMEM((B, H), F32),
            pltpu.VMEM((8, B), F32),
            pltpu.VMEM((8, B), F32),
            pltpu.VMEM((B, H), F32),
            pltpu.VMEM((8, NCODES), F32),
        ],
    )(l2, a0, a1, a2, w0, w1, w2, cb2, wihT, bih2, whhT, bhh2,
      codebook, cbkT, wih2T, cbih2, whh2T, cbhh2, wf, bf, uf)

    return out


# speedup vs baseline: 2.3015x; 1.0225x over previous
import jax
import jax.numpy as jnp
from jax.experimental import pallas as pl
from jax.experimental.pallas import tpu as pltpu

B = 32
C = 39
L = 2048
D = 256
H = 256
K = 6
STRIDE = 2
NCODES = 1024
NHEADS = 4
ATT_H = 128

T = (L - K) // STRIDE + 1
TP = 1024
CK = C * STRIDE
CKP = 128

TCH = 64
NC = TP // TCH

F32 = jnp.float32


def _gru_gates(g, gh, h):
    r = jax.nn.sigmoid(g[:, :H] + gh[:, :H])
    z = jax.nn.sigmoid(g[:, H:2 * H] + gh[:, H:2 * H])
    n = jnp.tanh(g[:, 2 * H:] + r * gh[:, 2 * H:])
    return (1.0 - z) * n + z * h


def _fused(l_ref, a0_ref, a1_ref, a2_ref, w0_ref, w1_ref, w2_ref, cb_ref,
           wih_ref, bih_ref, whh_ref, bhh_ref,
           cbk_ref, cbkT_ref, wih2_ref, bih2_ref, whh2_ref, bhh2_ref,
           wf_ref, bf_ref, uf_ref,
           out_ref,
           h1_ref, h2_ref, pidx_ref, gi1_ref, gi2_ref, y_ref, hs_ref,
           cont_ref, idx_ref, n0_ref, n1_ref, n2_ref, n3_ref, den_ref,
           scst_ref, hsst_ref, cn_ref):
    s = pl.program_id(0)
    nums = [n0_ref, n1_ref, n2_ref, n3_ref]

    @pl.when(s == 0)
    def _():
        h1_ref[...] = jnp.zeros_like(h1_ref)
        cbT0 = cbkT_ref[...]
        cn_ref[0:1] = jnp.sum(cbT0 * cbT0, axis=0, keepdims=True)

    @pl.when(s == 1)
    def _():
        h2_ref[...] = jnp.zeros_like(h2_ref)
        pidx_ref[...] = jnp.full_like(pidx_ref, -1)
        for nr in nums:
            nr[...] = jnp.zeros_like(nr)
        den_ref[...] = jnp.zeros_like(den_ref)

    a0 = a0_ref[...].reshape(TCH * B, CKP)
    a1 = a1_ref[...].reshape(TCH * B, CKP)
    a2 = a2_ref[...].reshape(TCH * B, CKP)
    x = (jnp.dot(a0, w0_ref[...], preferred_element_type=F32)
         + jnp.dot(a1, w1_ref[...], preferred_element_type=F32)
         + jnp.dot(a2, w2_ref[...], preferred_element_type=F32)
         + cb_ref[...])
    gi1 = jnp.dot(x, wih_ref[...], preferred_element_type=F32) + bih_ref[...]
    gi1_ref[...] = gi1.reshape(TCH, B, 3 * H)

    @pl.when(s >= 1)
    def _():
        yf = y_ref[...].reshape(TCH * B, H)
        d = (cn_ref[0:1]
             - 2.0 * jnp.dot(yf, cbkT_ref[...], preferred_element_type=F32))
        d3 = d.reshape(TCH, B, NCODES)
        dmin = jnp.min(d3, axis=2, keepdims=True)
        iota = jax.lax.broadcasted_iota(jnp.int32, (TCH, B, NCODES), 2)
        cand = jnp.where(d3 == dmin, iota, NCODES)
        cmin = jnp.min(cand, axis=2, keepdims=True)
        idx3 = cmin[:, :, 0]
        idx_ref[...] = idx3
        onehot = (cand == cmin).astype(F32).reshape(TCH * B, NCODES)
        q = jnp.dot(onehot, cbk_ref[...], preferred_element_type=F32)
        gi2 = (jnp.dot(q, wih2_ref[...], preferred_element_type=F32)
               + bih2_ref[...])
        gi2_ref[...] = gi2.reshape(TCH, B, 3 * H)

        @pl.when(s >= 2)
        def _():
            nsd = (idx3[0:1] != pidx_ref[0:1]).astype(F32)
            hstash = hsst_ref[...]
            for k in range(NHEADS):
                wd = jnp.exp(scst_ref[k]) * nsd[0]
                nums[k][...] += wd[:, None] * hstash
                den_ref[k] += wd
            den_ref[NHEADS] += nsd[0]

        prev = jnp.concatenate([pidx_ref[0:1], idx3[:TCH - 1]], axis=0)
        cont3 = (idx3 == prev).astype(F32)[:, :, None]
        cont_ref[...] = jnp.broadcast_to(cont3, (TCH, B, H))
        pidx_ref[0:1] = idx3[TCH - 1:]

    whh1 = whh_ref[...]
    bhh1 = bhh_ref[...]
    whh2 = whh2_ref[...]
    bhh2 = bhh2_ref[...]

    def body(k, carry):
        h1, h2 = carry
        g1 = gi1_ref[pl.ds(k, 1)][0]
        gh1 = jnp.dot(h1, whh1, preferred_element_type=F32) + bhh1
        hn1 = _gru_gates(g1, gh1, h1)
        y_ref[pl.ds(k, 1)] = hn1[None]

        g2 = gi2_ref[pl.ds(k, 1)][0]
        c2 = cont_ref[pl.ds(k, 1)][0]
        hin = h2 * c2
        gh2 = jnp.dot(hin, whh2, preferred_element_type=F32) + bhh2
        hn2 = _gru_gates(g2, gh2, hin)
        hs_ref[pl.ds(k, 1)] = hn2[None]
        return hn1, hn2

    h1f, h2f = jax.lax.fori_loop(0, TCH, body,
                                 (h1_ref[...], h2_ref[...]), unroll=8)
    h1_ref[...] = h1f
    h2_ref[...] = h2f

    lc3 = ((l_ref[...] - (K - STRIDE)) // STRIDE)[None, :, :]
    tt3 = jax.lax.broadcasted_iota(jnp.int32, (TCH, B, 1), 0) + s * TCH
    y_ref[...] = y_ref[...] * (tt3 < lc3).astype(F32)

    @pl.when(s >= 1)
    def _():
        idx3 = idx_ref[...]
        hs3 = hs_ref[...]
        hsf = hs3.reshape(TCH * B, H)
        e = jnp.tanh(jnp.dot(hsf, wf_ref[...], preferred_element_type=F32)
                     + bf_ref[...])
        ew3 = (e * uf_ref[...]).reshape(TCH, B, NHEADS * ATT_H)
        tglob = (jax.lax.broadcasted_iota(jnp.int32, (TCH, B), 0)
                 + (s - 1) * TCH)
        nxt = jnp.concatenate([idx3[1:], idx3[TCH - 1:]], axis=0)
        nsr = jnp.logical_or(nxt != idx3, tglob == T - 1)
        m3 = jnp.logical_and(nsr, tglob < T).astype(F32)
        for k in range(NHEADS):
            sk = jnp.sum(ew3[:, :, k * ATT_H:(k + 1) * ATT_H], axis=2)
            wk = jnp.exp(sk) * m3
            nums[k][...] += jnp.sum(wk[:, :, None] * hs3, axis=0)
            den_ref[k] += jnp.sum(wk, axis=0)
            scst_ref[k:k + 1] = sk[TCH - 1:]
        den_ref[NHEADS] += jnp.sum(m3, axis=0)
        hsst_ref[...] = hs3[TCH - 1]

    @pl.when(s == NC)
    def _():
        ew0 = jnp.tanh(bf_ref[...]) * uf_ref[...]
        nseg = den_ref[NHEADS]
        ss = jnp.zeros((B, 1), F32)
        ps = []
        for k in range(NHEADS):
            s0 = jnp.sum(ew0[0, k * ATT_H:(k + 1) * ATT_H])
            dk = den_ref[k] + (float(T) - nseg) * jnp.exp(s0)
            pk = nums[k][...] / dk[:, None]
            ps.append(pk)
            ss = ss + jnp.sum(pk * pk, axis=1, keepdims=True)
        nrm = jnp.maximum(jnp.sqrt(ss), 1e-12)
        for k in range(NHEADS):
            out_ref[:, k * H:(k + 1) * H] = ps[k] / nrm


def kernel(input, l, conv_w, conv_b, gru_Wih, gru_Whh, gru_bih, gru_bhh,
           cell_Wih, cell_Whh, cell_bih, cell_bhh, codebook, att_W, att_b,
           att_u):
    p = input.reshape(B, C, L // STRIDE, STRIDE)
    p = p.transpose(2, 0, 1, 3).reshape(L // STRIDE, B, CK)
    p = jnp.pad(p, ((0, 0), (0, 0), (0, CKP - CK)))
    a0 = jnp.pad(p, ((0, TCH), (0, 0), (0, 0)))
    a1 = jnp.pad(p[1:], ((0, TCH + 1), (0, 0), (0, 0)))
    a2 = jnp.pad(p[2:], ((0, TCH + 2), (0, 0), (0, 0)))
    ws = []
    for j in range(K // STRIDE):
        wj = conv_w[:, :, STRIDE * j:STRIDE * (j + 1)].reshape(D, CK).T
        ws.append(jnp.pad(wj, ((0, CKP - CK), (0, 0))))
    w0, w1, w2 = ws
    l2 = l.astype(jnp.int32).reshape(B, 1)
    cb2 = conv_b.reshape(1, D)
    wihT = gru_Wih.T
    whhT = gru_Whh.T
    bih2 = gru_bih.reshape(1, 3 * H)
    bhh2 = gru_bhh.reshape(1, 3 * H)
    wih2T = cell_Wih.T
    whh2T = cell_Whh.T
    cbih2 = cell_bih.reshape(1, 3 * H)
    cbhh2 = cell_bhh.reshape(1, 3 * H)
    cbkT = codebook.T
    wf = att_W.transpose(1, 0, 2).reshape(H, NHEADS * ATT_H)
    bf = att_b.reshape(1, NHEADS * ATT_H)
    uf = att_u.reshape(1, NHEADS * ATT_H)

    cnst = lambda i: (0, 0)
    out = pl.pallas_call(
        _fused,
        grid=(NC + 1,),
        in_specs=[
            pl.BlockSpec((B, 1), cnst),
            pl.BlockSpec((TCH, B, CKP), lambda i: (i, 0, 0)),
            pl.BlockSpec((TCH, B, CKP), lambda i: (i, 0, 0)),
            pl.BlockSpec((TCH, B, CKP), lambda i: (i, 0, 0)),
            pl.BlockSpec((CKP, D), cnst),
            pl.BlockSpec((CKP, D), cnst),
            pl.BlockSpec((CKP, D), cnst),
            pl.BlockSpec((1, D), cnst),
            pl.BlockSpec((D, 3 * H), cnst),
            pl.BlockSpec((1, 3 * H), cnst),
            pl.BlockSpec((H, 3 * H), cnst),
            pl.BlockSpec((1, 3 * H), cnst),
            pl.BlockSpec((NCODES, H), cnst),
            pl.BlockSpec((H, NCODES), cnst),
            pl.BlockSpec((H, 3 * H), cnst),
            pl.BlockSpec((1, 3 * H), cnst),
            pl.BlockSpec((H, 3 * H), cnst),
            pl.BlockSpec((1, 3 * H), cnst),
            pl.BlockSpec((H, NHEADS * ATT_H), cnst),
            pl.BlockSpec((1, NHEADS * ATT_H), cnst),
            pl.BlockSpec((1, NHEADS * ATT_H), cnst),
        ],
        out_specs=pl.BlockSpec((B, NHEADS * H), cnst),
        out_shape=jax.ShapeDtypeStruct((B, NHEADS * H), F32),
        scratch_shapes=[
            pltpu.VMEM((B, H), F32),
            pltpu.VMEM((B, H), F32),
            pltpu.VMEM((8, B), jnp.int32),
            pltpu.VMEM((TCH, B, 3 * H), F32),
            pltpu.VMEM((TCH, B, 3 * H), F32),
            pltpu.VMEM((TCH, B, H), F32),
            pltpu.VMEM((TCH, B, H), F32),
            pltpu.VMEM((TCH, B, H), F32),
            pltpu.VMEM((TCH, B), jnp.int32),
            pltpu.VMEM((B, H), F32),
            pltpu.VMEM((B, H), F32),
            pltpu.VMEM((B, H), F32),
            pltpu.VMEM((B, H), F32),
            pltpu.VMEM((8, B), F32),
            pltpu.VMEM((8, B), F32),
            pltpu.VMEM((B, H), F32),
            pltpu.VMEM((8, NCODES), F32),
        ],
    )(l2, a0, a1, a2, w0, w1, w2, cb2, wihT, bih2, whhT, bhh2,
      codebook, cbkT, wih2T, cbih2, whh2T, cbhh2, wf, bf, uf)

    return out


# speedup vs baseline: 2.3285x; 1.0117x over previous
import jax
import jax.numpy as jnp
from jax.experimental import pallas as pl
from jax.experimental.pallas import tpu as pltpu

B = 32
C = 39
L = 2048
D = 256
H = 256
K = 6
STRIDE = 2
NCODES = 1024
NHEADS = 4
ATT_H = 128

T = (L - K) // STRIDE + 1
TP = 1024
CK = C * STRIDE
CKP = 128

TCH = 64
NC = TP // TCH

F32 = jnp.float32


def _gru_gates(g, gh, h):
    r = jax.nn.sigmoid(g[:, :H] + gh[:, :H])
    z = jax.nn.sigmoid(g[:, H:2 * H] + gh[:, H:2 * H])
    n = jnp.tanh(g[:, 2 * H:] + r * gh[:, 2 * H:])
    return (1.0 - z) * n + z * h


def _fused(l_ref, a0_ref, a1_ref, a2_ref, w0_ref, w1_ref, w2_ref, cb_ref,
           wih_ref, bih_ref, whh_ref, bhh_ref,
           cbk_ref, cbkT_ref, wih2_ref, bih2_ref, whh2_ref, bhh2_ref,
           wf_ref, bf_ref, uf_ref,
           out_ref,
           h1_ref, h2_ref, pidx_ref, gi1_ref, gi2_ref, y_ref, hs_ref,
           cont_ref, idx_ref, n0_ref, n1_ref, n2_ref, n3_ref, den_ref,
           scst_ref, hsst_ref, cn_ref):
    s = pl.program_id(0)
    nums = [n0_ref, n1_ref, n2_ref, n3_ref]

    @pl.when(s == 0)
    def _():
        h1_ref[...] = jnp.zeros_like(h1_ref)
        cbT0 = cbkT_ref[...]
        cn_ref[0:1] = jnp.sum(cbT0 * cbT0, axis=0, keepdims=True)

    @pl.when(s == 1)
    def _():
        h2_ref[...] = jnp.zeros_like(h2_ref)
        pidx_ref[...] = jnp.full_like(pidx_ref, -1)
        for nr in nums:
            nr[...] = jnp.zeros_like(nr)
        den_ref[...] = jnp.zeros_like(den_ref)

    a0 = a0_ref[...].reshape(TCH * B, CKP)
    a1 = a1_ref[...].reshape(TCH * B, CKP)
    a2 = a2_ref[...].reshape(TCH * B, CKP)
    x = (jnp.dot(a0, w0_ref[...], preferred_element_type=F32)
         + jnp.dot(a1, w1_ref[...], preferred_element_type=F32)
         + jnp.dot(a2, w2_ref[...], preferred_element_type=F32)
         + cb_ref[...])
    gi1 = jnp.dot(x, wih_ref[...], preferred_element_type=F32) + bih_ref[...]
    gi1_ref[...] = gi1.reshape(TCH, B, 3 * H)

    @pl.when(s >= 1)
    def _():
        yf = y_ref[...].reshape(TCH * B, H)
        d = (cn_ref[0:1]
             - 2.0 * jnp.dot(yf, cbkT_ref[...], preferred_element_type=F32))
        d3 = d.reshape(TCH, B, NCODES)
        dmin = jnp.min(d3, axis=2, keepdims=True)
        iota = jax.lax.broadcasted_iota(jnp.int32, (TCH, B, NCODES), 2)
        cand = jnp.where(d3 == dmin, iota, NCODES)
        cmin = jnp.min(cand, axis=2, keepdims=True)
        idx3 = cmin[:, :, 0]
        idx_ref[...] = idx3
        onehot = (cand == cmin).astype(F32).reshape(TCH * B, NCODES)
        q = jnp.dot(onehot, cbk_ref[...], preferred_element_type=F32)
        gi2 = (jnp.dot(q, wih2_ref[...], preferred_element_type=F32)
               + bih2_ref[...])
        gi2_ref[...] = gi2.reshape(TCH, B, 3 * H)

        @pl.when(s >= 2)
        def _():
            nsd = (idx3[0:1] != pidx_ref[0:1]).astype(F32)
            hstash = hsst_ref[...]
            for k in range(NHEADS):
                wd = jnp.exp(scst_ref[k]) * nsd[0]
                nums[k][...] += wd[:, None] * hstash
                den_ref[k] += wd
            den_ref[NHEADS] += nsd[0]

        prev = jnp.concatenate([pidx_ref[0:1], idx3[:TCH - 1]], axis=0)
        cont3 = (idx3 == prev).astype(F32)[:, :, None]
        cont_ref[...] = jnp.broadcast_to(cont3, (TCH, B, H))
        pidx_ref[0:1] = idx3[TCH - 1:]

    whh1 = whh_ref[...]
    bhh1 = bhh_ref[...]
    whh2 = whh2_ref[...]
    bhh2 = bhh2_ref[...]

    def body(k, carry):
        h1, h2 = carry
        g1 = gi1_ref[pl.ds(k, 1)][0]
        gh1 = jnp.dot(h1, whh1, preferred_element_type=F32) + bhh1
        hn1 = _gru_gates(g1, gh1, h1)
        y_ref[pl.ds(k, 1)] = hn1[None]

        g2 = gi2_ref[pl.ds(k, 1)][0]
        c2 = cont_ref[pl.ds(k, 1)][0]
        hin = h2 * c2
        gh2 = jnp.dot(hin, whh2, preferred_element_type=F32) + bhh2
        hn2 = _gru_gates(g2, gh2, hin)
        hs_ref[pl.ds(k, 1)] = hn2[None]
        return hn1, hn2

    h1f, h2f = jax.lax.fori_loop(0, TCH, body,
                                 (h1_ref[...], h2_ref[...]), unroll=16)
    h1_ref[...] = h1f
    h2_ref[...] = h2f

    lc3 = ((l_ref[...] - (K - STRIDE)) // STRIDE)[None, :, :]
    tt3 = jax.lax.broadcasted_iota(jnp.int32, (TCH, B, 1), 0) + s * TCH
    y_ref[...] = y_ref[...] * (tt3 < lc3).astype(F32)

    @pl.when(s >= 1)
    def _():
        idx3 = idx_ref[...]
        hs3 = hs_ref[...]
        hsf = hs3.reshape(TCH * B, H)
        e = jnp.tanh(jnp.dot(hsf, wf_ref[...], preferred_element_type=F32)
                     + bf_ref[...])
        ew3 = (e * uf_ref[...]).reshape(TCH, B, NHEADS * ATT_H)
        tglob = (jax.lax.broadcasted_iota(jnp.int32, (TCH, B), 0)
                 + (s - 1) * TCH)
        nxt = jnp.concatenate([idx3[1:], idx3[TCH - 1:]], axis=0)
        nsr = jnp.logical_or(nxt != idx3, tglob == T - 1)
        m3 = jnp.logical_and(nsr, tglob < T).astype(F32)
        for k in range(NHEADS):
            sk = jnp.sum(ew3[:, :, k * ATT_H:(k + 1) * ATT_H], axis=2)
            wk = jnp.exp(sk) * m3
            nums[k][...] += jnp.sum(wk[:, :, None] * hs3, axis=0)
            den_ref[k] += jnp.sum(wk, axis=0)
            scst_ref[k:k + 1] = sk[TCH - 1:]
        den_ref[NHEADS] += jnp.sum(m3, axis=0)
        hsst_ref[...] = hs3[TCH - 1]

    @pl.when(s == NC)
    def _():
        ew0 = jnp.tanh(bf_ref[...]) * uf_ref[...]
        nseg = den_ref[NHEADS]
        ss = jnp.zeros((B, 1), F32)
        ps = []
        for k in range(NHEADS):
            s0 = jnp.sum(ew0[0, k * ATT_H:(k + 1) * ATT_H])
            dk = den_ref[k] + (float(T) - nseg) * jnp.exp(s0)
            pk = nums[k][...] / dk[:, None]
            ps.append(pk)
            ss = ss + jnp.sum(pk * pk, axis=1, keepdims=True)
        nrm = jnp.maximum(jnp.sqrt(ss), 1e-12)
        for k in range(NHEADS):
            out_ref[:, k * H:(k + 1) * H] = ps[k] / nrm


def kernel(input, l, conv_w, conv_b, gru_Wih, gru_Whh, gru_bih, gru_bhh,
           cell_Wih, cell_Whh, cell_bih, cell_bhh, codebook, att_W, att_b,
           att_u):
    p = input.reshape(B, C, L // STRIDE, STRIDE)
    p = p.transpose(2, 0, 1, 3).reshape(L // STRIDE, B, CK)
    p = jnp.pad(p, ((0, 0), (0, 0), (0, CKP - CK)))
    a0 = jnp.pad(p, ((0, TCH), (0, 0), (0, 0)))
    a1 = jnp.pad(p[1:], ((0, TCH + 1), (0, 0), (0, 0)))
    a2 = jnp.pad(p[2:], ((0, TCH + 2), (0, 0), (0, 0)))
    ws = []
    for j in range(K // STRIDE):
        wj = conv_w[:, :, STRIDE * j:STRIDE * (j + 1)].reshape(D, CK).T
        ws.append(jnp.pad(wj, ((0, CKP - CK), (0, 0))))
    w0, w1, w2 = ws
    l2 = l.astype(jnp.int32).reshape(B, 1)
    cb2 = conv_b.reshape(1, D)
    wihT = gru_Wih.T
    whhT = gru_Whh.T
    bih2 = gru_bih.reshape(1, 3 * H)
    bhh2 = gru_bhh.reshape(1, 3 * H)
    wih2T = cell_Wih.T
    whh2T = cell_Whh.T
    cbih2 = cell_bih.reshape(1, 3 * H)
    cbhh2 = cell_bhh.reshape(1, 3 * H)
    cbkT = codebook.T
    wf = att_W.transpose(1, 0, 2).reshape(H, NHEADS * ATT_H)
    bf = att_b.reshape(1, NHEADS * ATT_H)
    uf = att_u.reshape(1, NHEADS * ATT_H)

    cnst = lambda i: (0, 0)
    out = pl.pallas_call(
        _fused,
        grid=(NC + 1,),
        in_specs=[
            pl.BlockSpec((B, 1), cnst),
            pl.BlockSpec((TCH, B, CKP), lambda i: (i, 0, 0)),
            pl.BlockSpec((TCH, B, CKP), lambda i: (i, 0, 0)),
            pl.BlockSpec((TCH, B, CKP), lambda i: (i, 0, 0)),
            pl.BlockSpec((CKP, D), cnst),
            pl.BlockSpec((CKP, D), cnst),
            pl.BlockSpec((CKP, D), cnst),
            pl.BlockSpec((1, D), cnst),
            pl.BlockSpec((D, 3 * H), cnst),
            pl.BlockSpec((1, 3 * H), cnst),
            pl.BlockSpec((H, 3 * H), cnst),
            pl.BlockSpec((1, 3 * H), cnst),
            pl.BlockSpec((NCODES, H), cnst),
            pl.BlockSpec((H, NCODES), cnst),
            pl.BlockSpec((H, 3 * H), cnst),
            pl.BlockSpec((1, 3 * H), cnst),
            pl.BlockSpec((H, 3 * H), cnst),
            pl.BlockSpec((1, 3 * H), cnst),
            pl.BlockSpec((H, NHEADS * ATT_H), cnst),
            pl.BlockSpec((1, NHEADS * ATT_H), cnst),
            pl.BlockSpec((1, NHEADS * ATT_H), cnst),
        ],
        out_specs=pl.BlockSpec((B, NHEADS * H), cnst),
        out_shape=jax.ShapeDtypeStruct((B, NHEADS * H), F32),
        scratch_shapes=[
            pltpu.VMEM((B, H), F32),
            pltpu.VMEM((B, H), F32),
            pltpu.VMEM((8, B), jnp.int32),
            pltpu.VMEM((TCH, B, 3 * H), F32),
            pltpu.VMEM((TCH, B, 3 * H), F32),
            pltpu.VMEM((TCH, B, H), F32),
            pltpu.VMEM((TCH, B, H), F32),
            pltpu.VMEM((TCH, B, H), F32),
            pltpu.VMEM((TCH, B), jnp.int32),
            pltpu.VMEM((B, H), F32),
            pltpu.VMEM((B, H), F32),
            pltpu.VMEM((B, H), F32),
            pltpu.VMEM((B, H), F32),
            pltpu.VMEM((8, B), F32),
            pltpu.VMEM((8, B), F32),
            pltpu.VMEM((B, H), F32),
            pltpu.VMEM((8, NCODES), F32),
        ],
    )(l2, a0, a1, a2, w0, w1, w2, cb2, wihT, bih2, whhT, bhh2,
      codebook, cbkT, wih2T, cbih2, whh2T, cbhh2, wf, bf, uf)

    return out


# speedup vs baseline: 2.3303x; 1.0008x over previous
import jax
import jax.numpy as jnp
from jax.experimental import pallas as pl
from jax.experimental.pallas import tpu as pltpu

B = 32
C = 39
L = 2048
D = 256
H = 256
K = 6
STRIDE = 2
NCODES = 1024
NHEADS = 4
ATT_H = 128

T = (L - K) // STRIDE + 1
TP = 1024
CK = C * STRIDE
CKP = 128

TCH = 64
NC = TP // TCH

F32 = jnp.float32


def _gru_gates(g, gh, h):
    r = jax.nn.sigmoid(g[:, :H] + gh[:, :H])
    z = jax.nn.sigmoid(g[:, H:2 * H] + gh[:, H:2 * H])
    n = jnp.tanh(g[:, 2 * H:] + r * gh[:, 2 * H:])
    return (1.0 - z) * n + z * h


def _fused(l_ref, a0_ref, a1_ref, a2_ref, w0_ref, w1_ref, w2_ref, cb_ref,
           wih_ref, bih_ref, whh_ref, bhh_ref,
           cbk_ref, cbkT_ref, wih2_ref, bih2_ref, whh2_ref, bhh2_ref,
           wf_ref, bf_ref, uf_ref,
           out_ref,
           h1_ref, h2_ref, pidx_ref, gi1_ref, gi2_ref, y_ref, hs_ref,
           cont_ref, idx_ref, n0_ref, n1_ref, n2_ref, n3_ref, den_ref,
           scst_ref, hsst_ref, cn_ref):
    s = pl.program_id(0)
    nums = [n0_ref, n1_ref, n2_ref, n3_ref]

    @pl.when(s == 0)
    def _():
        h1_ref[...] = jnp.zeros_like(h1_ref)
        cbT0 = cbkT_ref[...]
        cn_ref[0:1] = jnp.sum(cbT0 * cbT0, axis=0, keepdims=True)

    @pl.when(s == 1)
    def _():
        h2_ref[...] = jnp.zeros_like(h2_ref)
        pidx_ref[...] = jnp.full_like(pidx_ref, -1)
        for nr in nums:
            nr[...] = jnp.zeros_like(nr)
        den_ref[...] = jnp.zeros_like(den_ref)

    yf = y_ref[...].reshape(TCH * B, H)
    d = (cn_ref[0:1]
         - 2.0 * jnp.dot(yf, cbkT_ref[...], preferred_element_type=F32))
    d3 = d.reshape(TCH, B, NCODES)
    dmin = jnp.min(d3, axis=2, keepdims=True)
    iota = jax.lax.broadcasted_iota(jnp.int32, (TCH, B, NCODES), 2)
    cand = jnp.where(d3 == dmin, iota, NCODES)
    cmin = jnp.min(cand, axis=2, keepdims=True)
    idx3 = cmin[:, :, 0]
    idx_ref[...] = idx3
    onehot = (cand == cmin).astype(F32).reshape(TCH * B, NCODES)
    q = jnp.dot(onehot, cbk_ref[...], preferred_element_type=F32)
    gi2 = (jnp.dot(q, wih2_ref[...], preferred_element_type=F32)
           + bih2_ref[...])
    gi2_ref[...] = gi2.reshape(TCH, B, 3 * H)
    prev = jnp.concatenate([pidx_ref[0:1], idx3[:TCH - 1]], axis=0)
    cont3 = (idx3 == prev).astype(F32)[:, :, None]
    cont_ref[...] = jnp.broadcast_to(cont3, (TCH, B, H))
    pidx_ref[0:1] = idx3[TCH - 1:]

    a0 = a0_ref[...].reshape(TCH * B, CKP)
    a1 = a1_ref[...].reshape(TCH * B, CKP)
    a2 = a2_ref[...].reshape(TCH * B, CKP)
    x = (jnp.dot(a0, w0_ref[...], preferred_element_type=F32)
         + jnp.dot(a1, w1_ref[...], preferred_element_type=F32)
         + jnp.dot(a2, w2_ref[...], preferred_element_type=F32)
         + cb_ref[...])
    gi1 = jnp.dot(x, wih_ref[...], preferred_element_type=F32) + bih_ref[...]
    gi1_ref[...] = gi1.reshape(TCH, B, 3 * H)

    @pl.when(s >= 2)
    def _():
        nsd = (idx3[0:1] != prev[0:1]).astype(F32)
        hstash = hsst_ref[...]
        for k in range(NHEADS):
            wd = jnp.exp(scst_ref[k]) * nsd[0]
            nums[k][...] += wd[:, None] * hstash
            den_ref[k] += wd
        den_ref[NHEADS] += nsd[0]

    whh1 = whh_ref[...]
    bhh1 = bhh_ref[...]
    whh2 = whh2_ref[...]
    bhh2 = bhh2_ref[...]

    def body(k, carry):
        h1, h2 = carry
        g1 = gi1_ref[pl.ds(k, 1)][0]
        gh1 = jnp.dot(h1, whh1, preferred_element_type=F32) + bhh1
        hn1 = _gru_gates(g1, gh1, h1)
        y_ref[pl.ds(k, 1)] = hn1[None]

        g2 = gi2_ref[pl.ds(k, 1)][0]
        c2 = cont_ref[pl.ds(k, 1)][0]
        hin = h2 * c2
        gh2 = jnp.dot(hin, whh2, preferred_element_type=F32) + bhh2
        hn2 = _gru_gates(g2, gh2, hin)
        hs_ref[pl.ds(k, 1)] = hn2[None]
        return hn1, hn2

    h1f, h2f = jax.lax.fori_loop(0, TCH, body,
                                 (h1_ref[...], h2_ref[...]), unroll=16)
    h1_ref[...] = h1f
    h2_ref[...] = h2f

    lc3 = ((l_ref[...] - (K - STRIDE)) // STRIDE)[None, :, :]
    tt3 = jax.lax.broadcasted_iota(jnp.int32, (TCH, B, 1), 0) + s * TCH
    y_ref[...] = y_ref[...] * (tt3 < lc3).astype(F32)

    hs3 = hs_ref[...]
    hsf = hs3.reshape(TCH * B, H)
    e = jnp.tanh(jnp.dot(hsf, wf_ref[...], preferred_element_type=F32)
                 + bf_ref[...])
    ew3 = (e * uf_ref[...]).reshape(TCH, B, NHEADS * ATT_H)
    tglob = (jax.lax.broadcasted_iota(jnp.int32, (TCH, B), 0)
             + (s - 1) * TCH)
    nxt = jnp.concatenate([idx3[1:], idx3[TCH - 1:]], axis=0)
    nsr = jnp.logical_or(nxt != idx3, tglob == T - 1)
    mok = jnp.logical_and(nsr, tglob < T)
    m3 = jnp.logical_and(mok, s >= 1).astype(F32)
    for k in range(NHEADS):
        sk = jnp.sum(ew3[:, :, k * ATT_H:(k + 1) * ATT_H], axis=2)
        wk = jnp.exp(sk) * m3
        nums[k][...] += jnp.sum(wk[:, :, None] * hs3, axis=0)
        den_ref[k] += jnp.sum(wk, axis=0)
        scst_ref[k:k + 1] = sk[TCH - 1:]
    den_ref[NHEADS] += jnp.sum(m3, axis=0)
    hsst_ref[...] = hs3[TCH - 1]

    @pl.when(s == NC)
    def _():
        ew0 = jnp.tanh(bf_ref[...]) * uf_ref[...]
        nseg = den_ref[NHEADS]
        ss = jnp.zeros((B, 1), F32)
        ps = []
        for k in range(NHEADS):
            s0 = jnp.sum(ew0[0, k * ATT_H:(k + 1) * ATT_H])
            dk = den_ref[k] + (float(T) - nseg) * jnp.exp(s0)
            pk = nums[k][...] / dk[:, None]
            ps.append(pk)
            ss = ss + jnp.sum(pk * pk, axis=1, keepdims=True)
        nrm = jnp.maximum(jnp.sqrt(ss), 1e-12)
        for k in range(NHEADS):
            out_ref[:, k * H:(k + 1) * H] = ps[k] / nrm


def kernel(input, l, conv_w, conv_b, gru_Wih, gru_Whh, gru_bih, gru_bhh,
           cell_Wih, cell_Whh, cell_bih, cell_bhh, codebook, att_W, att_b,
           att_u):
    p = input.reshape(B, C, L // STRIDE, STRIDE)
    p = p.transpose(2, 0, 1, 3).reshape(L // STRIDE, B, CK)
    p = jnp.pad(p, ((0, 0), (0, 0), (0, CKP - CK)))
    a0 = jnp.pad(p, ((0, TCH), (0, 0), (0, 0)))
    a1 = jnp.pad(p[1:], ((0, TCH + 1), (0, 0), (0, 0)))
    a2 = jnp.pad(p[2:], ((0, TCH + 2), (0, 0), (0, 0)))
    ws = []
    for j in range(K // STRIDE):
        wj = conv_w[:, :, STRIDE * j:STRIDE * (j + 1)].reshape(D, CK).T
        ws.append(jnp.pad(wj, ((0, CKP - CK), (0, 0))))
    w0, w1, w2 = ws
    l2 = l.astype(jnp.int32).reshape(B, 1)
    cb2 = conv_b.reshape(1, D)
    wihT = gru_Wih.T
    whhT = gru_Whh.T
    bih2 = gru_bih.reshape(1, 3 * H)
    bhh2 = gru_bhh.reshape(1, 3 * H)
    wih2T = cell_Wih.T
    whh2T = cell_Whh.T
    cbih2 = cell_bih.reshape(1, 3 * H)
    cbhh2 = cell_bhh.reshape(1, 3 * H)
    cbkT = codebook.T
    wf = att_W.transpose(1, 0, 2).reshape(H, NHEADS * ATT_H)
    bf = att_b.reshape(1, NHEADS * ATT_H)
    uf = att_u.reshape(1, NHEADS * ATT_H)

    cnst = lambda i: (0, 0)
    out = pl.pallas_call(
        _fused,
        grid=(NC + 1,),
        in_specs=[
            pl.BlockSpec((B, 1), cnst),
            pl.BlockSpec((TCH, B, CKP), lambda i: (i, 0, 0)),
            pl.BlockSpec((TCH, B, CKP), lambda i: (i, 0, 0)),
            pl.BlockSpec((TCH, B, CKP), lambda i: (i, 0, 0)),
            pl.BlockSpec((CKP, D), cnst),
            pl.BlockSpec((CKP, D), cnst),
            pl.BlockSpec((CKP, D), cnst),
            pl.BlockSpec((1, D), cnst),
            pl.BlockSpec((D, 3 * H), cnst),
            pl.BlockSpec((1, 3 * H), cnst),
            pl.BlockSpec((H, 3 * H), cnst),
            pl.BlockSpec((1, 3 * H), cnst),
            pl.BlockSpec((NCODES, H), cnst),
            pl.BlockSpec((H, NCODES), cnst),
            pl.BlockSpec((H, 3 * H), cnst),
            pl.BlockSpec((1, 3 * H), cnst),
            pl.BlockSpec((H, 3 * H), cnst),
            pl.BlockSpec((1, 3 * H), cnst),
            pl.BlockSpec((H, NHEADS * ATT_H), cnst),
            pl.BlockSpec((1, NHEADS * ATT_H), cnst),
            pl.BlockSpec((1, NHEADS * ATT_H), cnst),
        ],
        out_specs=pl.BlockSpec((B, NHEADS * H), cnst),
        out_shape=jax.ShapeDtypeStruct((B, NHEADS * H), F32),
        scratch_shapes=[
            pltpu.VMEM((B, H), F32),
            pltpu.VMEM((B, H), F32),
            pltpu.VMEM((8, B), jnp.int32),
            pltpu.VMEM((TCH, B, 3 * H), F32),
            pltpu.VMEM((TCH, B, 3 * H), F32),
            pltpu.VMEM((TCH, B, H), F32),
            pltpu.VMEM((TCH, B, H), F32),
            pltpu.VMEM((TCH, B, H), F32),
            pltpu.VMEM((TCH, B), jnp.int32),
            pltpu.VMEM((B, H), F32),
            pltpu.VMEM((B, H), F32),
            pltpu.VMEM((B, H), F32),
            pltpu.VMEM((B, H), F32),
            pltpu.VMEM((8, B), F32),
            pltpu.VMEM((8, B), F32),
            pltpu.VMEM((B, H), F32),
            pltpu.VMEM((8, NCODES), F32),
        ],
    )(l2, a0, a1, a2, w0, w1, w2, cb2, wihT, bih2, whhT, bhh2,
      codebook, cbkT, wih2T, cbih2, whh2T, cbhh2, wf, bf, uf)

    return out
